# Initial kernel scaffold; baseline (speedup 1.0000x reference)
import functools

import jax
import jax.numpy as jnp
from jax import lax
from jax.experimental import pallas as pl
from jax.experimental.pallas import tpu as pltpu
from jax.experimental.pallas import tpu_sc as plsc

N = 10000
E = 320000
D_IN = 128
D_EDGE = 16
D_MID = 128

NC = 2
NS = 16
L = 16
NW = NC * NS

NP = 10240
NPT = NP // NS
EP = 327680
KB = 128
EW = EP // NW
NB = EW // KB
ET = EP // NS
NB2 = ET // KB

_f32 = jnp.float32
_i32 = jnp.int32

_MESH = plsc.VectorSubcoreMesh(core_axis_name="c", subcore_axis_name="s")


def _leaky_exp(lg):
    lg = jnp.where(lg >= 0, lg, 0.2 * lg)
    return jnp.exp(lg)


@functools.partial(
    pl.kernel,
    out_type=[
        jax.ShapeDtypeStruct((NC, NP, 16), _f32),
        jax.ShapeDtypeStruct((NC, NP), _f32),
    ],
    mesh=_MESH,
    scratch_types=[
        pltpu.VMEM((NP,), _f32),
        pltpu.VMEM((NP,), _f32),
        pltpu.VMEM((NP,), _f32),
        pltpu.VMEM((KB,), _i32),
        pltpu.VMEM((KB,), _i32),
        pltpu.VMEM((KB,), _f32),
        pltpu.VMEM((KB,), _f32),
        pltpu.VMEM((KB, 16), _f32),
        pltpu.VMEM((NS, NPT), _f32),
        pltpu.VMEM_SHARED((NP, 16), _f32),
        pltpu.VMEM_SHARED((NP, 16), _f32),
        pltpu.VMEM_SHARED((NS, NP), _f32),
        pltpu.SemaphoreType.DMA,
    ],
)
def _sc_layer16(src_hbm, dst_hbm, eg_hbm, s_hbm, t_hbm, h_hbm, z16_hbm,
                z1_hbm, acc_out, den_out,
                s_tab, t_tab, den_tab, srcv, dstv, egv, exv, rows, den_red,
                h_sh, acc_sh, den_slots, sem):
    c = lax.axis_index("c")
    sid = lax.axis_index("s")
    wid = c * NS + sid
    rows_sl = pl.ds(sid * NPT, NPT)

    pltpu.sync_copy(s_hbm, s_tab)
    pltpu.sync_copy(t_hbm, t_tab)
    pltpu.sync_copy(z1_hbm, den_tab)
    pltpu.sync_copy(h_hbm.at[rows_sl, :], h_sh.at[rows_sl, :])
    pltpu.sync_copy(z16_hbm.at[rows_sl, :], acc_sh.at[rows_sl, :])
    plsc.subcore_barrier()

    base = wid * EW

    @pl.loop(0, NB)
    def _batch(b):
        off = base + b * KB
        pltpu.sync_copy(src_hbm.at[pl.ds(off, KB)], srcv)
        pltpu.sync_copy(dst_hbm.at[pl.ds(off, KB)], dstv)
        pltpu.sync_copy(eg_hbm.at[pl.ds(off, KB)], egv)
        for g in range(KB // L):
            sl = pl.ds(g * L, L)
            sg = srcv[sl]
            dg = dstv[sl]
            ex = _leaky_exp(plsc.load_gather(s_tab, [sg])
                            + plsc.load_gather(t_tab, [dg]) + egv[sl])
            exv[sl] = ex
            plsc.addupdate_scatter(den_tab, [dg], ex)
        pltpu.async_copy(h_sh.at[srcv], rows, sem).wait()

        @pl.loop(0, KB)
        def _scale(e):
            exs = plsc.load_gather(exv, [jnp.full((L,), e, _i32)])
            rows[e, :] = rows[e, :] * exs

        pltpu.async_copy(rows, acc_sh.at[dstv], sem, add=True).wait()

    plsc.subcore_barrier()
    pltpu.sync_copy(acc_sh.at[rows_sl, :], acc_out.at[c, rows_sl, :])
    pltpu.sync_copy(den_tab, den_slots.at[sid])
    plsc.subcore_barrier()
    pltpu.sync_copy(den_slots.at[:, rows_sl], den_red)
    for g in range(NPT // L):
        sl = pl.ds(g * L, L)
        v = den_red[0, sl]
        for k in range(1, NS):
            v = v + den_red[k, sl]
        den_red[0, sl] = v
    pltpu.sync_copy(den_red.at[0], den_out.at[c, rows_sl])


@functools.partial(
    pl.kernel,
    out_type=[
        jax.ShapeDtypeStruct((NC, NP, 64), _f32),
        jax.ShapeDtypeStruct((NP,), _f32),
    ],
    mesh=_MESH,
    scratch_types=[
        pltpu.VMEM((NP,), _f32),
        pltpu.VMEM((NP,), _f32),
        pltpu.VMEM((NP,), _f32),
        pltpu.VMEM((KB,), _i32),
        pltpu.VMEM((KB,), _i32),
        pltpu.VMEM((KB,), _f32),
        pltpu.VMEM((KB,), _f32),
        pltpu.VMEM((KB, 64), _f32),
        pltpu.VMEM((NS, NPT), _f32),
        pltpu.VMEM_SHARED((NP, 64), _f32),
        pltpu.VMEM_SHARED((NP, 64), _f32),
        pltpu.VMEM_SHARED((NS, NP), _f32),
        pltpu.SemaphoreType.DMA,
    ],
)
def _sc_layer128(src_hbm, dst_hbm, eg_hbm, s_hbm, t_hbm, h_hbm, z64_hbm,
                 z1_hbm, acc_out, den_out,
                 s_tab, t_tab, den_tab, srcv, dstv, egv, exv, rows, den_red,
                 h_sh, acc_sh, den_slots, sem):
    c = lax.axis_index("c")
    sid = lax.axis_index("s")
    rows_sl = pl.ds(sid * NPT, NPT)
    col0 = c * 64

    pltpu.sync_copy(s_hbm, s_tab)
    pltpu.sync_copy(t_hbm, t_tab)
    pltpu.sync_copy(z1_hbm, den_tab)
    pltpu.sync_copy(h_hbm.at[rows_sl, pl.ds(col0, 64)], h_sh.at[rows_sl, :])
    pltpu.sync_copy(z64_hbm.at[rows_sl, :], acc_sh.at[rows_sl, :])
    plsc.subcore_barrier()

    base = sid * ET

    @pl.loop(0, NB2)
    def _batch(b):
        off = base + b * KB
        pltpu.sync_copy(src_hbm.at[pl.ds(off, KB)], srcv)
        pltpu.sync_copy(dst_hbm.at[pl.ds(off, KB)], dstv)
        pltpu.sync_copy(eg_hbm.at[pl.ds(off, KB)], egv)
        for g in range(KB // L):
            sl = pl.ds(g * L, L)
            sg = srcv[sl]
            dg = dstv[sl]
            ex = _leaky_exp(plsc.load_gather(s_tab, [sg])
                            + plsc.load_gather(t_tab, [dg]) + egv[sl])
            exv[sl] = ex

            @pl.when(c == 0)
            def _():
                plsc.addupdate_scatter(den_tab, [dg], ex)

        pltpu.async_copy(h_sh.at[srcv], rows, sem).wait()

        @pl.loop(0, KB)
        def _scale(e):
            exs = plsc.load_gather(exv, [jnp.full((L,), e, _i32)])
            for k in range(64 // L):
                sl = pl.ds(k * L, L)
                rows[e, sl] = rows[e, sl] * exs

        pltpu.async_copy(rows, acc_sh.at[dstv], sem, add=True).wait()

    plsc.subcore_barrier()
    pltpu.sync_copy(acc_sh.at[rows_sl, :], acc_out.at[c, rows_sl, :])

    @pl.when(c == 0)
    def _den_epilogue():
        pltpu.sync_copy(den_tab, den_slots.at[sid])
        plsc.subcore_barrier()
        pltpu.sync_copy(den_slots.at[:, rows_sl], den_red)
        for g in range(NPT // L):
            sl = pl.ds(g * L, L)
            v = den_red[0, sl]
            for k in range(1, NS):
                v = v + den_red[k, sl]
            den_red[0, sl] = v
        pltpu.sync_copy(den_red.at[0], den_out.at[rows_sl])


def _tc_pre_body(x_ref, w1_ref, asrc_ref, adst_ref, h1_ref, s1_ref, t1_ref):
    h1 = jnp.dot(x_ref[...], w1_ref[...], preferred_element_type=_f32)
    h1_ref[...] = h1
    s1_ref[...] = h1 @ asrc_ref[...]
    t1_ref[...] = h1 @ adst_ref[...]


_tc_pre = pl.pallas_call(
    _tc_pre_body,
    out_shape=[
        jax.ShapeDtypeStruct((NP, 16), _f32),
        jax.ShapeDtypeStruct((NP,), _f32),
        jax.ShapeDtypeStruct((NP,), _f32),
    ],
)

_EGB = 4096


def _tc_eg_body(ea_ref, we1_ref, a1e_ref, we2_ref, a2e_ref, we3_ref, a3e_ref,
                eg1_ref, eg2_ref, eg3_ref):
    ve1 = we1_ref[...] @ a1e_ref[...]
    ve2 = we2_ref[...] @ a2e_ref[...]
    ve3 = we3_ref[...] @ a3e_ref[...]
    ea = ea_ref[...]
    eg1_ref[...] = ea @ ve1
    eg2_ref[...] = ea @ ve2
    eg3_ref[...] = ea @ ve3


_tc_eg = pl.pallas_call(
    _tc_eg_body,
    grid=(EP // _EGB,),
    in_specs=[
        pl.BlockSpec((_EGB, D_EDGE), lambda i: (i, 0)),
        pl.BlockSpec((D_EDGE, D_EDGE), lambda i: (0, 0)),
        pl.BlockSpec((D_EDGE,), lambda i: (0,)),
        pl.BlockSpec((D_EDGE, D_MID), lambda i: (0, 0)),
        pl.BlockSpec((D_MID,), lambda i: (0,)),
        pl.BlockSpec((D_EDGE, 2), lambda i: (0, 0)),
        pl.BlockSpec((2,), lambda i: (0,)),
    ],
    out_specs=[
        pl.BlockSpec((_EGB,), lambda i: (i,)),
        pl.BlockSpec((_EGB,), lambda i: (i,)),
        pl.BlockSpec((_EGB,), lambda i: (i,)),
    ],
    out_shape=[
        jax.ShapeDtypeStruct((EP,), _f32),
        jax.ShapeDtypeStruct((EP,), _f32),
        jax.ShapeDtypeStruct((EP,), _f32),
    ],
)


def _tc_post1_body(acc_ref, den_ref, b1_ref, w2_ref, asrc_ref, adst_ref,
                   h2_ref, s2_ref, t2_ref):
    d = den_ref[0, :] + den_ref[1, :]
    inv = 1.0 / (d + 1e-16)
    out1 = (acc_ref[0] + acc_ref[1]) * inv[:, None] + b1_ref[...][None, :]
    x2 = jnp.maximum(out1, 0.0)
    h2 = jnp.dot(x2, w2_ref[...], preferred_element_type=_f32)
    h2_ref[...] = h2
    s2_ref[...] = h2 @ asrc_ref[...]
    t2_ref[...] = h2 @ adst_ref[...]


_tc_post1 = pl.pallas_call(
    _tc_post1_body,
    out_shape=[
        jax.ShapeDtypeStruct((NP, D_MID), _f32),
        jax.ShapeDtypeStruct((NP,), _f32),
        jax.ShapeDtypeStruct((NP,), _f32),
    ],
)


def _tc_post2_body(acc_ref, den_ref, b2_ref, w3p_ref, asrc_ref, adst_ref,
                   h3_ref, s3_ref, t3_ref):
    inv = 1.0 / (den_ref[...] + 1e-16)
    out2 = jnp.concatenate([acc_ref[0], acc_ref[1]], axis=1)
    out2 = out2 * inv[:, None] + b2_ref[...][None, :]
    h3 = jnp.dot(out2, w3p_ref[...], preferred_element_type=_f32)
    h3_ref[...] = h3
    s3_ref[...] = h3 @ asrc_ref[...]
    t3_ref[...] = h3 @ adst_ref[...]


_tc_post2 = pl.pallas_call(
    _tc_post2_body,
    out_shape=[
        jax.ShapeDtypeStruct((NP, 16), _f32),
        jax.ShapeDtypeStruct((NP,), _f32),
        jax.ShapeDtypeStruct((NP,), _f32),
    ],
)


def _tc_post3_body(acc_ref, den_ref, b3_ref, wc_ref, bc_ref, out_ref):
    d = den_ref[0, :] + den_ref[1, :]
    inv = 1.0 / (d + 1e-16)
    out3 = (acc_ref[0] + acc_ref[1])[:, :2] * inv[:, None] + b3_ref[...][None, :]
    logits = jnp.dot(out3, wc_ref[...], preferred_element_type=_f32)
    logits = logits + bc_ref[...][None, :]
    row = lax.broadcasted_iota(_i32, (NP, 1), 0)
    valid = row < N
    neg = jnp.float32(-jnp.inf)
    masked = jnp.where(valid, logits, neg)
    mx = jnp.max(masked, axis=0, keepdims=True)
    z = jnp.where(valid, jnp.exp(logits - mx), 0.0)
    lse = jnp.log(jnp.sum(z, axis=0, keepdims=True)) + mx
    out_ref[...] = logits - lse


_tc_post3 = pl.pallas_call(
    _tc_post3_body,
    out_shape=jax.ShapeDtypeStruct((NP, 21), _f32),
)


def kernel(x, edge_index, edge_attr, W1, We1, a1_src, a1_dst, a1_edge, b1,
           W2, We2, a2_src, a2_dst, a2_edge, b2,
           W3, We3, a3_src, a3_dst, a3_edge, b3, Wc, bc):
    x_p = jnp.pad(x, ((0, NP - N), (0, 0)))
    src = jnp.pad(edge_index[0], (0, EP - E), constant_values=N)
    dst = jnp.pad(edge_index[1], (0, EP - E), constant_values=N)
    ea_p = jnp.pad(edge_attr, ((0, EP - E), (0, 0)))
    w3p = jnp.pad(W3, ((0, 0), (0, 16 - 2)))
    a3s = jnp.pad(a3_src, (0, 16 - 2))
    a3d = jnp.pad(a3_dst, (0, 16 - 2))

    z1 = jnp.zeros((NP,), _f32)
    z16 = jnp.zeros((NP, 16), _f32)
    z64 = jnp.zeros((NP, 64), _f32)

    h1, s1, t1 = _tc_pre(x_p, W1, a1_src, a1_dst)
    eg1, eg2, eg3 = _tc_eg(ea_p, We1, a1_edge, We2, a2_edge, We3, a3_edge)

    acc1, den1 = _sc_layer16(src, dst, eg1, s1, t1, h1, z16, z1)
    h2, s2, t2 = _tc_post1(acc1, den1, b1, W2, a2_src, a2_dst)

    acc2, den2 = _sc_layer128(src, dst, eg2, s2, t2, h2, z64, z1)
    h3, s3, t3 = _tc_post2(acc2, den2, b2, w3p, a3s, a3d)

    acc3, den3 = _sc_layer16(src, dst, eg3, s3, t3, h3, z16, z1)
    res = _tc_post3(acc3, den3, b3, Wc, bc)
    return res[:N, :]


# speedup vs baseline: 18.4680x; 18.4680x over previous
import functools

import jax
import jax.numpy as jnp
from jax import lax
from jax.experimental import pallas as pl
from jax.experimental.pallas import tpu as pltpu
from jax.experimental.pallas import tpu_sc as plsc

N = 10000
E = 320000
D_IN = 128
D_EDGE = 16
D_MID = 128

NC = 2
NS = 16
L = 16
NW = NC * NS

NP = 10240
NPT = NP // NS
EP = 327680
KB = 128
EW = EP // NW
NB = EW // KB
ET = EP // NS
NB2 = ET // KB

_f32 = jnp.float32
_i32 = jnp.int32

_MESH = plsc.VectorSubcoreMesh(core_axis_name="c", subcore_axis_name="s")
_SC_PARAMS = pltpu.CompilerParams(
    needs_layout_passes=False, use_tc_tiling_on_sc=False)


def _leaky_exp(lg):
    lg = jnp.where(lg >= 0, lg, 0.2 * lg)
    return jnp.exp(lg)


@functools.partial(
    pl.kernel,
    out_type=[
        jax.ShapeDtypeStruct((NC, NP, 16), _f32),
        jax.ShapeDtypeStruct((NC, NS, NP), _f32),
    ],
    mesh=_MESH,
    scratch_types=[
        pltpu.VMEM((NP,), _f32),
        pltpu.VMEM((NP,), _f32),
        pltpu.VMEM((NP,), _f32),
        pltpu.VMEM((KB,), _i32),
        pltpu.VMEM((KB,), _i32),
        pltpu.VMEM((KB,), _f32),
        pltpu.VMEM((KB,), _f32),
        pltpu.VMEM((KB, 16), _f32),
        pltpu.VMEM_SHARED((NP, 16), _f32),
        pltpu.VMEM_SHARED((NP, 16), _f32),
        pltpu.SemaphoreType.DMA,
    ],
    compiler_params=_SC_PARAMS,
)
def _sc_layer16(src_hbm, dst_hbm, eg_hbm, s_hbm, t_hbm, h_hbm, z16_hbm,
                z1_hbm, acc_out, den_out,
                s_tab, t_tab, den_tab, srcv, dstv, egv, exv, rows,
                h_sh, acc_sh, sem):
    c = lax.axis_index("c")
    sid = lax.axis_index("s")
    wid = c * NS + sid
    rows_sl = pl.ds(sid * NPT, NPT)

    pltpu.sync_copy(s_hbm, s_tab)
    pltpu.sync_copy(t_hbm, t_tab)
    pltpu.sync_copy(z1_hbm, den_tab)
    pltpu.sync_copy(h_hbm.at[rows_sl, :], h_sh.at[rows_sl, :])
    pltpu.sync_copy(z16_hbm.at[rows_sl, :], acc_sh.at[rows_sl, :])
    plsc.subcore_barrier()

    base = wid * EW

    @pl.loop(0, NB)
    def _batch(b):
        off = base + b * KB
        pltpu.sync_copy(src_hbm.at[pl.ds(off, KB)], srcv)
        pltpu.sync_copy(dst_hbm.at[pl.ds(off, KB)], dstv)
        pltpu.sync_copy(eg_hbm.at[pl.ds(off, KB)], egv)
        for g in range(KB // L):
            sl = pl.ds(g * L, L)
            sg = srcv[sl]
            dg = dstv[sl]
            ex = _leaky_exp(plsc.load_gather(s_tab, [sg])
                            + plsc.load_gather(t_tab, [dg]) + egv[sl])
            exv[sl] = ex
            plsc.addupdate_scatter(den_tab, [dg], ex)
        pltpu.async_copy(h_sh.at[srcv], rows, sem).wait()

        @pl.loop(0, KB)
        def _scale(e):
            exs = plsc.load_gather(exv, [jnp.full((L,), e, _i32)])
            rows[e, :] = rows[e, :] * exs

        pltpu.async_copy(rows, acc_sh.at[dstv], sem, add=True).wait()

    plsc.subcore_barrier()
    pltpu.sync_copy(acc_sh.at[rows_sl, :], acc_out.at[c, rows_sl, :])
    pltpu.sync_copy(den_tab, den_out.at[c, sid])


@functools.partial(
    pl.kernel,
    out_type=[
        jax.ShapeDtypeStruct((NC, NP, 64), _f32),
        jax.ShapeDtypeStruct((NS, NP), _f32),
    ],
    mesh=_MESH,
    scratch_types=[
        pltpu.VMEM((NP,), _f32),
        pltpu.VMEM((NP,), _f32),
        pltpu.VMEM((NP,), _f32),
        pltpu.VMEM((KB,), _i32),
        pltpu.VMEM((KB,), _i32),
        pltpu.VMEM((KB,), _f32),
        pltpu.VMEM((KB,), _f32),
        pltpu.VMEM((KB, 64), _f32),
        pltpu.VMEM_SHARED((NP, 64), _f32),
        pltpu.VMEM_SHARED((NP, 64), _f32),
        pltpu.SemaphoreType.DMA,
    ],
    compiler_params=_SC_PARAMS,
)
def _sc_layer128(src_hbm, dst_hbm, eg_hbm, s_hbm, t_hbm, h_hbm, z64_hbm,
                 z1_hbm, acc_out, den_out,
                 s_tab, t_tab, den_tab, srcv, dstv, egv, exv, rows,
                 h_sh, acc_sh, sem):
    c = lax.axis_index("c")
    sid = lax.axis_index("s")
    rows_sl = pl.ds(sid * NPT, NPT)
    col0 = c * 64

    pltpu.sync_copy(s_hbm, s_tab)
    pltpu.sync_copy(t_hbm, t_tab)
    pltpu.sync_copy(z1_hbm, den_tab)
    pltpu.sync_copy(h_hbm.at[rows_sl, pl.ds(col0, 64)], h_sh.at[rows_sl, :])
    pltpu.sync_copy(z64_hbm.at[rows_sl, :], acc_sh.at[rows_sl, :])
    plsc.subcore_barrier()

    base = sid * ET

    @pl.loop(0, NB2)
    def _batch(b):
        off = base + b * KB
        pltpu.sync_copy(src_hbm.at[pl.ds(off, KB)], srcv)
        pltpu.sync_copy(dst_hbm.at[pl.ds(off, KB)], dstv)
        pltpu.sync_copy(eg_hbm.at[pl.ds(off, KB)], egv)
        for g in range(KB // L):
            sl = pl.ds(g * L, L)
            sg = srcv[sl]
            dg = dstv[sl]
            ex = _leaky_exp(plsc.load_gather(s_tab, [sg])
                            + plsc.load_gather(t_tab, [dg]) + egv[sl])
            exv[sl] = ex

            @pl.when(c == 0)
            def _():
                plsc.addupdate_scatter(den_tab, [dg], ex)

        pltpu.async_copy(h_sh.at[srcv], rows, sem).wait()

        @pl.loop(0, KB)
        def _scale(e):
            exs = plsc.load_gather(exv, [jnp.full((L,), e, _i32)])
            for k in range(64 // L):
                sl = pl.ds(k * L, L)
                rows[e, sl] = rows[e, sl] * exs

        pltpu.async_copy(rows, acc_sh.at[dstv], sem, add=True).wait()

    plsc.subcore_barrier()
    pltpu.sync_copy(acc_sh.at[rows_sl, :], acc_out.at[c, rows_sl, :])

    @pl.when(c == 0)
    def _den_epilogue():
        pltpu.sync_copy(den_tab, den_out.at[sid])


def _tc_pre_body(x_ref, w1_ref, asrc_ref, adst_ref, h1_ref, s1_ref, t1_ref):
    h1 = jnp.dot(x_ref[...], w1_ref[...], preferred_element_type=_f32)
    h1_ref[...] = h1
    s1_ref[...] = h1 @ asrc_ref[...]
    t1_ref[...] = h1 @ adst_ref[...]


_tc_pre = pl.pallas_call(
    _tc_pre_body,
    out_shape=[
        jax.ShapeDtypeStruct((NP, 16), _f32),
        jax.ShapeDtypeStruct((NP,), _f32),
        jax.ShapeDtypeStruct((NP,), _f32),
    ],
)

_EGB = 4096


def _tc_eg_body(ea_ref, we1_ref, a1e_ref, we2_ref, a2e_ref, we3_ref, a3e_ref,
                eg1_ref, eg2_ref, eg3_ref):
    ve1 = we1_ref[...] @ a1e_ref[...]
    ve2 = we2_ref[...] @ a2e_ref[...]
    ve3 = we3_ref[...] @ a3e_ref[...]
    ea = ea_ref[...]
    eg1_ref[...] = ea @ ve1
    eg2_ref[...] = ea @ ve2
    eg3_ref[...] = ea @ ve3


_tc_eg = pl.pallas_call(
    _tc_eg_body,
    grid=(EP // _EGB,),
    in_specs=[
        pl.BlockSpec((_EGB, D_EDGE), lambda i: (i, 0)),
        pl.BlockSpec((D_EDGE, D_EDGE), lambda i: (0, 0)),
        pl.BlockSpec((D_EDGE,), lambda i: (0,)),
        pl.BlockSpec((D_EDGE, D_MID), lambda i: (0, 0)),
        pl.BlockSpec((D_MID,), lambda i: (0,)),
        pl.BlockSpec((D_EDGE, 2), lambda i: (0, 0)),
        pl.BlockSpec((2,), lambda i: (0,)),
    ],
    out_specs=[
        pl.BlockSpec((_EGB,), lambda i: (i,)),
        pl.BlockSpec((_EGB,), lambda i: (i,)),
        pl.BlockSpec((_EGB,), lambda i: (i,)),
    ],
    out_shape=[
        jax.ShapeDtypeStruct((EP,), _f32),
        jax.ShapeDtypeStruct((EP,), _f32),
        jax.ShapeDtypeStruct((EP,), _f32),
    ],
)


def _tc_post1_body(acc_ref, den_ref, b1_ref, w2_ref, asrc_ref, adst_ref,
                   h2_ref, s2_ref, t2_ref):
    d = jnp.sum(den_ref[...], axis=0)
    inv = 1.0 / (d + 1e-16)
    out1 = (acc_ref[0] + acc_ref[1]) * inv[:, None] + b1_ref[...][None, :]
    x2 = jnp.maximum(out1, 0.0)
    h2 = jnp.dot(x2, w2_ref[...], preferred_element_type=_f32)
    h2_ref[...] = h2
    s2_ref[...] = h2 @ asrc_ref[...]
    t2_ref[...] = h2 @ adst_ref[...]


_tc_post1 = pl.pallas_call(
    _tc_post1_body,
    out_shape=[
        jax.ShapeDtypeStruct((NP, D_MID), _f32),
        jax.ShapeDtypeStruct((NP,), _f32),
        jax.ShapeDtypeStruct((NP,), _f32),
    ],
)


def _tc_post2_body(acc_ref, den_ref, b2_ref, w3p_ref, asrc_ref, adst_ref,
                   h3_ref, s3_ref, t3_ref):
    d = jnp.sum(den_ref[...], axis=0)
    inv = 1.0 / (d + 1e-16)
    out2 = jnp.concatenate([acc_ref[0], acc_ref[1]], axis=1)
    out2 = out2 * inv[:, None] + b2_ref[...][None, :]
    h3 = jnp.dot(out2, w3p_ref[...], preferred_element_type=_f32)
    h3_ref[...] = h3
    s3_ref[...] = h3 @ asrc_ref[...]
    t3_ref[...] = h3 @ adst_ref[...]


_tc_post2 = pl.pallas_call(
    _tc_post2_body,
    out_shape=[
        jax.ShapeDtypeStruct((NP, 16), _f32),
        jax.ShapeDtypeStruct((NP,), _f32),
        jax.ShapeDtypeStruct((NP,), _f32),
    ],
)


def _tc_post3_body(acc_ref, den_ref, b3_ref, wc_ref, bc_ref, out_ref):
    d = jnp.sum(den_ref[...], axis=0)
    inv = 1.0 / (d + 1e-16)
    out3 = (acc_ref[0] + acc_ref[1])[:, :2] * inv[:, None] + b3_ref[...][None, :]
    logits = jnp.dot(out3, wc_ref[...], preferred_element_type=_f32)
    logits = logits + bc_ref[...][None, :]
    row = lax.broadcasted_iota(_i32, (NP, 1), 0)
    valid = row < N
    neg = jnp.float32(-jnp.inf)
    masked = jnp.where(valid, logits, neg)
    mx = jnp.max(masked, axis=0, keepdims=True)
    z = jnp.where(valid, jnp.exp(logits - mx), 0.0)
    lse = jnp.log(jnp.sum(z, axis=0, keepdims=True)) + mx
    out_ref[...] = logits - lse


_tc_post3 = pl.pallas_call(
    _tc_post3_body,
    out_shape=jax.ShapeDtypeStruct((NP, 21), _f32),
)


def kernel(x, edge_index, edge_attr, W1, We1, a1_src, a1_dst, a1_edge, b1,
           W2, We2, a2_src, a2_dst, a2_edge, b2,
           W3, We3, a3_src, a3_dst, a3_edge, b3, Wc, bc):
    x_p = jnp.pad(x, ((0, NP - N), (0, 0)))
    src = jnp.pad(edge_index[0], (0, EP - E), constant_values=N)
    dst = jnp.pad(edge_index[1], (0, EP - E), constant_values=N)
    ea_p = jnp.pad(edge_attr, ((0, EP - E), (0, 0)))
    w3p = jnp.pad(W3, ((0, 0), (0, 16 - 2)))
    a3s = jnp.pad(a3_src, (0, 16 - 2))
    a3d = jnp.pad(a3_dst, (0, 16 - 2))

    z1 = jnp.zeros((NP,), _f32)
    z16 = jnp.zeros((NP, 16), _f32)
    z64 = jnp.zeros((NP, 64), _f32)

    h1, s1, t1 = _tc_pre(x_p, W1, a1_src, a1_dst)
    eg1, eg2, eg3 = _tc_eg(ea_p, We1, a1_edge, We2, a2_edge, We3, a3_edge)

    acc1, den1 = _sc_layer16(src, dst, eg1, s1, t1, h1, z16, z1)
    h2, s2, t2 = _tc_post1(acc1, den1.reshape(NW, NP), b1, W2, a2_src, a2_dst)

    acc2, den2 = _sc_layer128(src, dst, eg2, s2, t2, h2, z64, z1)
    h3, s3, t3 = _tc_post2(acc2, den2, b2, w3p, a3s, a3d)

    acc3, den3 = _sc_layer16(src, dst, eg3, s3, t3, h3, z16, z1)
    res = _tc_post3(acc3, den3.reshape(NW, NP), b3, Wc, bc)
    return res[:N, :]


# speedup vs baseline: 23.8938x; 1.2938x over previous
import functools

import jax
import jax.numpy as jnp
from jax import lax
from jax.experimental import pallas as pl
from jax.experimental.pallas import tpu as pltpu
from jax.experimental.pallas import tpu_sc as plsc

N = 10000
E = 320000
D_IN = 128
D_EDGE = 16
D_MID = 128

NC = 2
NS = 16
L = 16
NW = NC * NS

NP = 10240
NPT = NP // NS
EP = 327680
KB = 128
EW = EP // NW
NB = EW // KB
ET = EP // NS
CH = 2048
NCH = ET // CH
NBC = CH // KB

_f32 = jnp.float32
_i32 = jnp.int32

_MESH = plsc.VectorSubcoreMesh(core_axis_name="c", subcore_axis_name="s")
_SC_PARAMS = pltpu.CompilerParams(
    needs_layout_passes=False, use_tc_tiling_on_sc=False)


def _leaky_exp(lg):
    lg = jnp.where(lg >= 0, lg, 0.2 * lg)
    return jnp.exp(lg)


@functools.partial(
    pl.kernel,
    out_type=[
        jax.ShapeDtypeStruct((NC, NP, 16), _f32),
        jax.ShapeDtypeStruct((NC, NS, NP), _f32),
    ],
    mesh=_MESH,
    scratch_types=[
        pltpu.VMEM((NP,), _f32),
        pltpu.VMEM((NP,), _f32),
        pltpu.VMEM((NP,), _f32),
        pltpu.VMEM((EW,), _i32),
        pltpu.VMEM((EW,), _i32),
        pltpu.VMEM((EW,), _f32),
        pltpu.VMEM((KB,), _f32),
        pltpu.VMEM((KB, 16), _f32),
        pltpu.VMEM_SHARED((NP, 16), _f32),
        pltpu.VMEM_SHARED((NP, 16), _f32),
        pltpu.SemaphoreType.DMA,
    ],
    compiler_params=_SC_PARAMS,
)
def _sc_layer16(src_hbm, dst_hbm, eg_hbm, s_hbm, t_hbm, h_hbm, z16_hbm,
                z1_hbm, acc_out, den_out,
                s_tab, t_tab, den_tab, srcv, dstv, egv, exv, rows,
                h_sh, acc_sh, sem):
    c = lax.axis_index("c")
    sid = lax.axis_index("s")
    wid = c * NS + sid
    rows_sl = pl.ds(sid * NPT, NPT)

    base = wid * EW
    pltpu.sync_copy(s_hbm, s_tab)
    pltpu.sync_copy(t_hbm, t_tab)
    pltpu.sync_copy(z1_hbm, den_tab)
    pltpu.sync_copy(src_hbm.at[pl.ds(base, EW)], srcv)
    pltpu.sync_copy(dst_hbm.at[pl.ds(base, EW)], dstv)
    pltpu.sync_copy(eg_hbm.at[pl.ds(base, EW)], egv)
    pltpu.sync_copy(h_hbm.at[rows_sl, :], h_sh.at[rows_sl, :])
    pltpu.sync_copy(z16_hbm.at[rows_sl, :], acc_sh.at[rows_sl, :])
    plsc.subcore_barrier()

    @pl.loop(0, NB)
    def _batch(b):
        off = b * KB
        for g in range(KB // L):
            sl = pl.ds(off + g * L, L)
            sg = srcv[sl]
            dg = dstv[sl]
            ex = _leaky_exp(plsc.load_gather(s_tab, [sg])
                            + plsc.load_gather(t_tab, [dg]) + egv[sl])
            exv[pl.ds(g * L, L)] = ex
            plsc.addupdate_scatter(den_tab, [dg], ex)
        bsl = pl.ds(off, KB)
        pltpu.async_copy(h_sh.at[srcv.at[bsl]], rows, sem).wait()

        @pl.loop(0, KB)
        def _scale(e):
            exs = plsc.load_gather(exv, [jnp.full((L,), e, _i32)])
            rows[e, :] = rows[e, :] * exs

        pltpu.async_copy(rows, acc_sh.at[dstv.at[bsl]], sem, add=True).wait()

    plsc.subcore_barrier()
    pltpu.sync_copy(acc_sh.at[rows_sl, :], acc_out.at[c, rows_sl, :])
    pltpu.sync_copy(den_tab, den_out.at[c, sid])


@functools.partial(
    pl.kernel,
    out_type=[
        jax.ShapeDtypeStruct((NC, NP, 64), _f32),
        jax.ShapeDtypeStruct((NS, NP), _f32),
    ],
    mesh=_MESH,
    scratch_types=[
        pltpu.VMEM((NP,), _f32),
        pltpu.VMEM((NP,), _f32),
        pltpu.VMEM((NP,), _f32),
        pltpu.VMEM((CH,), _i32),
        pltpu.VMEM((CH,), _i32),
        pltpu.VMEM((CH,), _f32),
        pltpu.VMEM((KB,), _f32),
        pltpu.VMEM((KB, 64), _f32),
        pltpu.VMEM_SHARED((NP, 64), _f32),
        pltpu.VMEM_SHARED((NP, 64), _f32),
        pltpu.SemaphoreType.DMA,
    ],
    compiler_params=_SC_PARAMS,
)
def _sc_layer128(src_hbm, dst_hbm, eg_hbm, s_hbm, t_hbm, h_hbm, z64_hbm,
                 z1_hbm, acc_out, den_out,
                 s_tab, t_tab, den_tab, srcv, dstv, egv, exv, rows,
                 h_sh, acc_sh, sem):
    c = lax.axis_index("c")
    sid = lax.axis_index("s")
    rows_sl = pl.ds(sid * NPT, NPT)
    col0 = c * 64

    base = sid * ET
    pltpu.sync_copy(s_hbm, s_tab)
    pltpu.sync_copy(t_hbm, t_tab)
    pltpu.sync_copy(z1_hbm, den_tab)
    pltpu.sync_copy(h_hbm.at[rows_sl, pl.ds(col0, 64)], h_sh.at[rows_sl, :])
    pltpu.sync_copy(z64_hbm.at[rows_sl, :], acc_sh.at[rows_sl, :])
    plsc.subcore_barrier()

    @pl.loop(0, NCH)
    def _chunk(ci):
        coff = base + ci * CH
        pltpu.sync_copy(src_hbm.at[pl.ds(coff, CH)], srcv)
        pltpu.sync_copy(dst_hbm.at[pl.ds(coff, CH)], dstv)
        pltpu.sync_copy(eg_hbm.at[pl.ds(coff, CH)], egv)

        @pl.loop(0, NBC)
        def _batch(b):
            off = b * KB
            for g in range(KB // L):
                sl = pl.ds(off + g * L, L)
                sg = srcv[sl]
                dg = dstv[sl]
                ex = _leaky_exp(plsc.load_gather(s_tab, [sg])
                                + plsc.load_gather(t_tab, [dg]) + egv[sl])
                exv[pl.ds(g * L, L)] = ex

                @pl.when(c == 0)
                def _():
                    plsc.addupdate_scatter(den_tab, [dg], ex)

            bsl = pl.ds(off, KB)
            pltpu.async_copy(h_sh.at[srcv.at[bsl]], rows, sem).wait()

            @pl.loop(0, KB)
            def _scale(e):
                exs = plsc.load_gather(exv, [jnp.full((L,), e, _i32)])
                for k in range(64 // L):
                    sl = pl.ds(k * L, L)
                    rows[e, sl] = rows[e, sl] * exs

            pltpu.async_copy(rows, acc_sh.at[dstv.at[bsl]], sem,
                             add=True).wait()

    plsc.subcore_barrier()
    pltpu.sync_copy(acc_sh.at[rows_sl, :], acc_out.at[c, rows_sl, :])

    @pl.when(c == 0)
    def _den_epilogue():
        pltpu.sync_copy(den_tab, den_out.at[sid])


def _tc_pre_body(x_ref, w1_ref, asrc_ref, adst_ref, h1_ref, s1_ref, t1_ref):
    h1 = jnp.dot(x_ref[...], w1_ref[...], preferred_element_type=_f32)
    h1_ref[...] = h1
    s1_ref[...] = h1 @ asrc_ref[...]
    t1_ref[...] = h1 @ adst_ref[...]


_tc_pre = pl.pallas_call(
    _tc_pre_body,
    out_shape=[
        jax.ShapeDtypeStruct((NP, 16), _f32),
        jax.ShapeDtypeStruct((NP,), _f32),
        jax.ShapeDtypeStruct((NP,), _f32),
    ],
)

_EGB = 4096


def _tc_eg_body(ea_ref, we1_ref, a1e_ref, we2_ref, a2e_ref, we3_ref, a3e_ref,
                eg1_ref, eg2_ref, eg3_ref):
    ve1 = we1_ref[...] @ a1e_ref[...]
    ve2 = we2_ref[...] @ a2e_ref[...]
    ve3 = we3_ref[...] @ a3e_ref[...]
    ea = ea_ref[...]
    eg1_ref[...] = ea @ ve1
    eg2_ref[...] = ea @ ve2
    eg3_ref[...] = ea @ ve3


_tc_eg = pl.pallas_call(
    _tc_eg_body,
    grid=(EP // _EGB,),
    in_specs=[
        pl.BlockSpec((_EGB, D_EDGE), lambda i: (i, 0)),
        pl.BlockSpec((D_EDGE, D_EDGE), lambda i: (0, 0)),
        pl.BlockSpec((D_EDGE,), lambda i: (0,)),
        pl.BlockSpec((D_EDGE, D_MID), lambda i: (0, 0)),
        pl.BlockSpec((D_MID,), lambda i: (0,)),
        pl.BlockSpec((D_EDGE, 2), lambda i: (0, 0)),
        pl.BlockSpec((2,), lambda i: (0,)),
    ],
    out_specs=[
        pl.BlockSpec((_EGB,), lambda i: (i,)),
        pl.BlockSpec((_EGB,), lambda i: (i,)),
        pl.BlockSpec((_EGB,), lambda i: (i,)),
    ],
    out_shape=[
        jax.ShapeDtypeStruct((EP,), _f32),
        jax.ShapeDtypeStruct((EP,), _f32),
        jax.ShapeDtypeStruct((EP,), _f32),
    ],
)


def _tc_post1_body(acc_ref, den_ref, b1_ref, w2_ref, asrc_ref, adst_ref,
                   h2_ref, s2_ref, t2_ref):
    d = jnp.sum(den_ref[...], axis=0)
    inv = 1.0 / (d + 1e-16)
    out1 = (acc_ref[0] + acc_ref[1]) * inv[:, None] + b1_ref[...][None, :]
    x2 = jnp.maximum(out1, 0.0)
    h2 = jnp.dot(x2, w2_ref[...], preferred_element_type=_f32)
    h2_ref[...] = h2
    s2_ref[...] = h2 @ asrc_ref[...]
    t2_ref[...] = h2 @ adst_ref[...]


_tc_post1 = pl.pallas_call(
    _tc_post1_body,
    out_shape=[
        jax.ShapeDtypeStruct((NP, D_MID), _f32),
        jax.ShapeDtypeStruct((NP,), _f32),
        jax.ShapeDtypeStruct((NP,), _f32),
    ],
)


def _tc_post2_body(acc_ref, den_ref, b2_ref, w3p_ref, asrc_ref, adst_ref,
                   h3_ref, s3_ref, t3_ref):
    d = jnp.sum(den_ref[...], axis=0)
    inv = 1.0 / (d + 1e-16)
    out2 = jnp.concatenate([acc_ref[0], acc_ref[1]], axis=1)
    out2 = out2 * inv[:, None] + b2_ref[...][None, :]
    h3 = jnp.dot(out2, w3p_ref[...], preferred_element_type=_f32)
    h3_ref[...] = h3
    s3_ref[...] = h3 @ asrc_ref[...]
    t3_ref[...] = h3 @ adst_ref[...]


_tc_post2 = pl.pallas_call(
    _tc_post2_body,
    out_shape=[
        jax.ShapeDtypeStruct((NP, 16), _f32),
        jax.ShapeDtypeStruct((NP,), _f32),
        jax.ShapeDtypeStruct((NP,), _f32),
    ],
)


def _tc_post3_body(acc_ref, den_ref, b3_ref, wc_ref, bc_ref, out_ref):
    d = jnp.sum(den_ref[...], axis=0)
    inv = 1.0 / (d + 1e-16)
    out3 = (acc_ref[0] + acc_ref[1])[:, :2] * inv[:, None] + b3_ref[...][None, :]
    logits = jnp.dot(out3, wc_ref[...], preferred_element_type=_f32)
    logits = logits + bc_ref[...][None, :]
    row = lax.broadcasted_iota(_i32, (NP, 1), 0)
    valid = row < N
    neg = jnp.float32(-jnp.inf)
    masked = jnp.where(valid, logits, neg)
    mx = jnp.max(masked, axis=0, keepdims=True)
    z = jnp.where(valid, jnp.exp(logits - mx), 0.0)
    lse = jnp.log(jnp.sum(z, axis=0, keepdims=True)) + mx
    out_ref[...] = logits - lse


_tc_post3 = pl.pallas_call(
    _tc_post3_body,
    out_shape=jax.ShapeDtypeStruct((NP, 21), _f32),
)


def kernel(x, edge_index, edge_attr, W1, We1, a1_src, a1_dst, a1_edge, b1,
           W2, We2, a2_src, a2_dst, a2_edge, b2,
           W3, We3, a3_src, a3_dst, a3_edge, b3, Wc, bc):
    x_p = jnp.pad(x, ((0, NP - N), (0, 0)))
    src = jnp.pad(edge_index[0], (0, EP - E), constant_values=N)
    dst = jnp.pad(edge_index[1], (0, EP - E), constant_values=N)
    ea_p = jnp.pad(edge_attr, ((0, EP - E), (0, 0)))
    w3p = jnp.pad(W3, ((0, 0), (0, 16 - 2)))
    a3s = jnp.pad(a3_src, (0, 16 - 2))
    a3d = jnp.pad(a3_dst, (0, 16 - 2))

    z1 = jnp.zeros((NP,), _f32)
    z16 = jnp.zeros((NP, 16), _f32)
    z64 = jnp.zeros((NP, 64), _f32)

    h1, s1, t1 = _tc_pre(x_p, W1, a1_src, a1_dst)
    eg1, eg2, eg3 = _tc_eg(ea_p, We1, a1_edge, We2, a2_edge, We3, a3_edge)

    acc1, den1 = _sc_layer16(src, dst, eg1, s1, t1, h1, z16, z1)
    h2, s2, t2 = _tc_post1(acc1, den1.reshape(NW, NP), b1, W2, a2_src, a2_dst)

    acc2, den2 = _sc_layer128(src, dst, eg2, s2, t2, h2, z64, z1)
    h3, s3, t3 = _tc_post2(acc2, den2, b2, w3p, a3s, a3d)

    acc3, den3 = _sc_layer16(src, dst, eg3, s3, t3, h3, z16, z1)
    res = _tc_post3(acc3, den3.reshape(NW, NP), b3, Wc, bc)
    return res[:N, :]


# speedup vs baseline: 27.8270x; 1.1646x over previous
import functools

import jax
import jax.numpy as jnp
from jax import lax
from jax.experimental import pallas as pl
from jax.experimental.pallas import tpu as pltpu
from jax.experimental.pallas import tpu_sc as plsc

N = 10000
E = 320000
D_IN = 128
D_EDGE = 16
D_MID = 128

NC = 2
NS = 16
L = 16
NW = NC * NS

NP = 10240
NPT = NP // NS
EP = 327680
KB = 128
EW = EP // NW
NB = EW // KB
ET = EP // NS
CH = 2048
NCH = ET // CH
NBC = CH // KB

_f32 = jnp.float32
_i32 = jnp.int32

_MESH = plsc.VectorSubcoreMesh(core_axis_name="c", subcore_axis_name="s")
_SC_PARAMS = pltpu.CompilerParams(
    needs_layout_passes=False, use_tc_tiling_on_sc=False)


def _leaky_exp(lg):
    lg = jnp.where(lg >= 0, lg, 0.2 * lg)
    return jnp.exp(lg)


@functools.partial(
    pl.kernel,
    out_type=[
        jax.ShapeDtypeStruct((NC, NP, 16), _f32),
        jax.ShapeDtypeStruct((NC, NS, NP), _f32),
    ],
    mesh=_MESH,
    scratch_types=[
        pltpu.VMEM((NP,), _f32),
        pltpu.VMEM((NP,), _f32),
        pltpu.VMEM((NP,), _f32),
        pltpu.VMEM((EW,), _i32),
        pltpu.VMEM((EW,), _i32),
        pltpu.VMEM((EW,), _f32),
        pltpu.VMEM((KB,), _f32),
        pltpu.VMEM((KB, 16), _f32),
        pltpu.VMEM_SHARED((NP, 16), _f32),
        pltpu.VMEM_SHARED((NP, 16), _f32),
        pltpu.SemaphoreType.DMA,
    ],
    compiler_params=_SC_PARAMS,
)
def _sc_layer16(src_hbm, dst_hbm, eg_hbm, s_hbm, t_hbm, h_hbm, z16_hbm,
                z1_hbm, acc_out, den_out,
                s_tab, t_tab, den_tab, srcv, dstv, egv, exv, rows,
                h_sh, acc_sh, sem):
    c = lax.axis_index("c")
    sid = lax.axis_index("s")
    wid = c * NS + sid
    rows_sl = pl.ds(sid * NPT, NPT)

    base = wid * EW
    pltpu.sync_copy(s_hbm, s_tab)
    pltpu.sync_copy(t_hbm, t_tab)
    pltpu.sync_copy(z1_hbm, den_tab)
    pltpu.sync_copy(src_hbm.at[pl.ds(base, EW)], srcv)
    pltpu.sync_copy(dst_hbm.at[pl.ds(base, EW)], dstv)
    pltpu.sync_copy(eg_hbm.at[pl.ds(base, EW)], egv)
    pltpu.sync_copy(h_hbm.at[rows_sl, :], h_sh.at[rows_sl, :])
    pltpu.sync_copy(z16_hbm.at[rows_sl, :], acc_sh.at[rows_sl, :])
    plsc.subcore_barrier()

    @pl.loop(0, NB)
    def _batch(b):
        off = b * KB
        for g in range(KB // L):
            sl = pl.ds(off + g * L, L)
            sg = srcv[sl]
            dg = dstv[sl]
            ex = _leaky_exp(plsc.load_gather(s_tab, [sg])
                            + plsc.load_gather(t_tab, [dg]) + egv[sl])
            exv[pl.ds(g * L, L)] = ex
            plsc.addupdate_scatter(den_tab, [dg], ex)
        bsl = pl.ds(off, KB)
        pltpu.async_copy(h_sh.at[srcv.at[bsl]], rows, sem).wait()

        @plsc.parallel_loop(0, KB, unroll=8)
        def _scale(e):
            exs = plsc.load_gather(exv, [jnp.full((L,), e, _i32)])
            rows[e, :] = rows[e, :] * exs

        pltpu.async_copy(rows, acc_sh.at[dstv.at[bsl]], sem, add=True).wait()

    plsc.subcore_barrier()
    pltpu.sync_copy(acc_sh.at[rows_sl, :], acc_out.at[c, rows_sl, :])
    pltpu.sync_copy(den_tab, den_out.at[c, sid])


@functools.partial(
    pl.kernel,
    out_type=[
        jax.ShapeDtypeStruct((NC, NP, 64), _f32),
        jax.ShapeDtypeStruct((NS, NP), _f32),
    ],
    mesh=_MESH,
    scratch_types=[
        pltpu.VMEM((NP,), _f32),
        pltpu.VMEM((NP,), _f32),
        pltpu.VMEM((NP,), _f32),
        pltpu.VMEM((CH,), _i32),
        pltpu.VMEM((CH,), _i32),
        pltpu.VMEM((CH,), _f32),
        pltpu.VMEM((KB,), _f32),
        pltpu.VMEM((KB, 64), _f32),
        pltpu.VMEM_SHARED((NP, 64), _f32),
        pltpu.VMEM_SHARED((NP, 64), _f32),
        pltpu.SemaphoreType.DMA,
    ],
    compiler_params=_SC_PARAMS,
)
def _sc_layer128(src_hbm, dst_hbm, eg_hbm, s_hbm, t_hbm, h_hbm, z64_hbm,
                 z1_hbm, acc_out, den_out,
                 s_tab, t_tab, den_tab, srcv, dstv, egv, exv, rows,
                 h_sh, acc_sh, sem):
    c = lax.axis_index("c")
    sid = lax.axis_index("s")
    rows_sl = pl.ds(sid * NPT, NPT)
    col0 = c * 64

    base = sid * ET
    pltpu.sync_copy(s_hbm, s_tab)
    pltpu.sync_copy(t_hbm, t_tab)
    pltpu.sync_copy(z1_hbm, den_tab)
    pltpu.sync_copy(h_hbm.at[rows_sl, pl.ds(col0, 64)], h_sh.at[rows_sl, :])
    pltpu.sync_copy(z64_hbm.at[rows_sl, :], acc_sh.at[rows_sl, :])
    plsc.subcore_barrier()

    @pl.loop(0, NCH)
    def _chunk(ci):
        coff = base + ci * CH
        pltpu.sync_copy(src_hbm.at[pl.ds(coff, CH)], srcv)
        pltpu.sync_copy(dst_hbm.at[pl.ds(coff, CH)], dstv)
        pltpu.sync_copy(eg_hbm.at[pl.ds(coff, CH)], egv)

        @pl.loop(0, NBC)
        def _batch(b):
            off = b * KB
            for g in range(KB // L):
                sl = pl.ds(off + g * L, L)
                sg = srcv[sl]
                dg = dstv[sl]
                ex = _leaky_exp(plsc.load_gather(s_tab, [sg])
                                + plsc.load_gather(t_tab, [dg]) + egv[sl])
                exv[pl.ds(g * L, L)] = ex

                @pl.when(c == 0)
                def _():
                    plsc.addupdate_scatter(den_tab, [dg], ex)

            bsl = pl.ds(off, KB)
            pltpu.async_copy(h_sh.at[srcv.at[bsl]], rows, sem).wait()

            @plsc.parallel_loop(0, KB, unroll=4)
            def _scale(e):
                exs = plsc.load_gather(exv, [jnp.full((L,), e, _i32)])
                for k in range(64 // L):
                    sl = pl.ds(k * L, L)
                    rows[e, sl] = rows[e, sl] * exs

            pltpu.async_copy(rows, acc_sh.at[dstv.at[bsl]], sem,
                             add=True).wait()

    plsc.subcore_barrier()
    pltpu.sync_copy(acc_sh.at[rows_sl, :], acc_out.at[c, rows_sl, :])

    @pl.when(c == 0)
    def _den_epilogue():
        pltpu.sync_copy(den_tab, den_out.at[sid])


def _tc_pre_body(x_ref, w1_ref, asrc_ref, adst_ref, h1_ref, s1_ref, t1_ref):
    h1 = jnp.dot(x_ref[...], w1_ref[...], preferred_element_type=_f32)
    h1_ref[...] = h1
    s1_ref[...] = h1 @ asrc_ref[...]
    t1_ref[...] = h1 @ adst_ref[...]


_tc_pre = pl.pallas_call(
    _tc_pre_body,
    out_shape=[
        jax.ShapeDtypeStruct((NP, 16), _f32),
        jax.ShapeDtypeStruct((NP,), _f32),
        jax.ShapeDtypeStruct((NP,), _f32),
    ],
)

_EGB = 4096


def _tc_eg_body(ea_ref, we1_ref, a1e_ref, we2_ref, a2e_ref, we3_ref, a3e_ref,
                eg1_ref, eg2_ref, eg3_ref):
    ve1 = we1_ref[...] @ a1e_ref[...]
    ve2 = we2_ref[...] @ a2e_ref[...]
    ve3 = we3_ref[...] @ a3e_ref[...]
    ea = ea_ref[...]
    eg1_ref[...] = ea @ ve1
    eg2_ref[...] = ea @ ve2
    eg3_ref[...] = ea @ ve3


_tc_eg = pl.pallas_call(
    _tc_eg_body,
    grid=(EP // _EGB,),
    in_specs=[
        pl.BlockSpec((_EGB, D_EDGE), lambda i: (i, 0)),
        pl.BlockSpec((D_EDGE, D_EDGE), lambda i: (0, 0)),
        pl.BlockSpec((D_EDGE,), lambda i: (0,)),
        pl.BlockSpec((D_EDGE, D_MID), lambda i: (0, 0)),
        pl.BlockSpec((D_MID,), lambda i: (0,)),
        pl.BlockSpec((D_EDGE, 2), lambda i: (0, 0)),
        pl.BlockSpec((2,), lambda i: (0,)),
    ],
    out_specs=[
        pl.BlockSpec((_EGB,), lambda i: (i,)),
        pl.BlockSpec((_EGB,), lambda i: (i,)),
        pl.BlockSpec((_EGB,), lambda i: (i,)),
    ],
    out_shape=[
        jax.ShapeDtypeStruct((EP,), _f32),
        jax.ShapeDtypeStruct((EP,), _f32),
        jax.ShapeDtypeStruct((EP,), _f32),
    ],
)


def _tc_post1_body(acc_ref, den_ref, b1_ref, w2_ref, asrc_ref, adst_ref,
                   h2_ref, s2_ref, t2_ref):
    d = jnp.sum(den_ref[...], axis=0)
    inv = 1.0 / (d + 1e-16)
    out1 = (acc_ref[0] + acc_ref[1]) * inv[:, None] + b1_ref[...][None, :]
    x2 = jnp.maximum(out1, 0.0)
    h2 = jnp.dot(x2, w2_ref[...], preferred_element_type=_f32)
    h2_ref[...] = h2
    s2_ref[...] = h2 @ asrc_ref[...]
    t2_ref[...] = h2 @ adst_ref[...]


_tc_post1 = pl.pallas_call(
    _tc_post1_body,
    out_shape=[
        jax.ShapeDtypeStruct((NP, D_MID), _f32),
        jax.ShapeDtypeStruct((NP,), _f32),
        jax.ShapeDtypeStruct((NP,), _f32),
    ],
)


def _tc_post2_body(acc_ref, den_ref, b2_ref, w3p_ref, asrc_ref, adst_ref,
                   h3_ref, s3_ref, t3_ref):
    d = jnp.sum(den_ref[...], axis=0)
    inv = 1.0 / (d + 1e-16)
    out2 = jnp.concatenate([acc_ref[0], acc_ref[1]], axis=1)
    out2 = out2 * inv[:, None] + b2_ref[...][None, :]
    h3 = jnp.dot(out2, w3p_ref[...], preferred_element_type=_f32)
    h3_ref[...] = h3
    s3_ref[...] = h3 @ asrc_ref[...]
    t3_ref[...] = h3 @ adst_ref[...]


_tc_post2 = pl.pallas_call(
    _tc_post2_body,
    out_shape=[
        jax.ShapeDtypeStruct((NP, 16), _f32),
        jax.ShapeDtypeStruct((NP,), _f32),
        jax.ShapeDtypeStruct((NP,), _f32),
    ],
)


def _tc_post3_body(acc_ref, den_ref, b3_ref, wc_ref, bc_ref, out_ref):
    d = jnp.sum(den_ref[...], axis=0)
    inv = 1.0 / (d + 1e-16)
    out3 = (acc_ref[0] + acc_ref[1])[:, :2] * inv[:, None] + b3_ref[...][None, :]
    logits = jnp.dot(out3, wc_ref[...], preferred_element_type=_f32)
    logits = logits + bc_ref[...][None, :]
    row = lax.broadcasted_iota(_i32, (NP, 1), 0)
    valid = row < N
    neg = jnp.float32(-jnp.inf)
    masked = jnp.where(valid, logits, neg)
    mx = jnp.max(masked, axis=0, keepdims=True)
    z = jnp.where(valid, jnp.exp(logits - mx), 0.0)
    lse = jnp.log(jnp.sum(z, axis=0, keepdims=True)) + mx
    out_ref[...] = logits - lse


_tc_post3 = pl.pallas_call(
    _tc_post3_body,
    out_shape=jax.ShapeDtypeStruct((NP, 21), _f32),
)


def kernel(x, edge_index, edge_attr, W1, We1, a1_src, a1_dst, a1_edge, b1,
           W2, We2, a2_src, a2_dst, a2_edge, b2,
           W3, We3, a3_src, a3_dst, a3_edge, b3, Wc, bc):
    x_p = jnp.pad(x, ((0, NP - N), (0, 0)))
    src = jnp.pad(edge_index[0], (0, EP - E), constant_values=N)
    dst = jnp.pad(edge_index[1], (0, EP - E), constant_values=N)
    ea_p = jnp.pad(edge_attr, ((0, EP - E), (0, 0)))
    w3p = jnp.pad(W3, ((0, 0), (0, 16 - 2)))
    a3s = jnp.pad(a3_src, (0, 16 - 2))
    a3d = jnp.pad(a3_dst, (0, 16 - 2))

    z1 = jnp.zeros((NP,), _f32)
    z16 = jnp.zeros((NP, 16), _f32)
    z64 = jnp.zeros((NP, 64), _f32)

    h1, s1, t1 = _tc_pre(x_p, W1, a1_src, a1_dst)
    eg1, eg2, eg3 = _tc_eg(ea_p, We1, a1_edge, We2, a2_edge, We3, a3_edge)

    acc1, den1 = _sc_layer16(src, dst, eg1, s1, t1, h1, z16, z1)
    h2, s2, t2 = _tc_post1(acc1, den1.reshape(NW, NP), b1, W2, a2_src, a2_dst)

    acc2, den2 = _sc_layer128(src, dst, eg2, s2, t2, h2, z64, z1)
    h3, s3, t3 = _tc_post2(acc2, den2, b2, w3p, a3s, a3d)

    acc3, den3 = _sc_layer16(src, dst, eg3, s3, t3, h3, z16, z1)
    res = _tc_post3(acc3, den3.reshape(NW, NP), b3, Wc, bc)
    return res[:N, :]


# speedup vs baseline: 28.8059x; 1.0352x over previous
import functools

import jax
import jax.numpy as jnp
from jax import lax
from jax.experimental import pallas as pl
from jax.experimental.pallas import tpu as pltpu
from jax.experimental.pallas import tpu_sc as plsc

N = 10000
E = 320000
D_IN = 128
D_EDGE = 16
D_MID = 128

NC = 2
NS = 16
L = 16
NW = NC * NS

NP = 10240
NPT = NP // NS
EP = 327680
KB = 128
EW = EP // NW
NB = EW // KB
ET = EP // NS
CH = 2048
NCH = ET // CH
NBC = CH // KB

_f32 = jnp.float32
_i32 = jnp.int32

_MESH = plsc.VectorSubcoreMesh(core_axis_name="c", subcore_axis_name="s")
_SC_PARAMS = pltpu.CompilerParams(
    needs_layout_passes=False, use_tc_tiling_on_sc=False)


def _leaky_exp(lg):
    lg = jnp.where(lg >= 0, lg, 0.2 * lg)
    return jnp.exp(lg)


@functools.partial(
    pl.kernel,
    out_type=[
        jax.ShapeDtypeStruct((NC, NP, 16), _f32),
        jax.ShapeDtypeStruct((NC, NS, NP), _f32),
    ],
    mesh=_MESH,
    scratch_types=[
        pltpu.VMEM((NP,), _f32),
        pltpu.VMEM((NP,), _f32),
        pltpu.VMEM((NP,), _f32),
        pltpu.VMEM((EW,), _i32),
        pltpu.VMEM((EW,), _i32),
        pltpu.VMEM((EW,), _f32),
        pltpu.VMEM((KB,), _f32),
        pltpu.VMEM((KB, 16), _f32),
        pltpu.VMEM_SHARED((NP, 16), _f32),
        pltpu.VMEM_SHARED((NP, 16), _f32),
        pltpu.SemaphoreType.DMA,
    ],
    compiler_params=_SC_PARAMS,
)
def _sc_layer16(src_hbm, dst_hbm, eg_hbm, s_hbm, t_hbm, h_hbm, z16_hbm,
                z1_hbm, acc_out, den_out,
                s_tab, t_tab, den_tab, srcv, dstv, egv, exv, rows,
                h_sh, acc_sh, sem):
    c = lax.axis_index("c")
    sid = lax.axis_index("s")
    wid = c * NS + sid
    rows_sl = pl.ds(sid * NPT, NPT)

    base = wid * EW
    pltpu.sync_copy(s_hbm, s_tab)
    pltpu.sync_copy(t_hbm, t_tab)
    pltpu.sync_copy(z1_hbm, den_tab)
    pltpu.sync_copy(src_hbm.at[pl.ds(base, EW)], srcv)
    pltpu.sync_copy(dst_hbm.at[pl.ds(base, EW)], dstv)
    pltpu.sync_copy(eg_hbm.at[pl.ds(base, EW)], egv)
    pltpu.sync_copy(h_hbm.at[rows_sl, :], h_sh.at[rows_sl, :])
    pltpu.sync_copy(z16_hbm.at[rows_sl, :], acc_sh.at[rows_sl, :])
    plsc.subcore_barrier()

    @pl.loop(0, NB)
    def _batch(b):
        off = b * KB
        for g in range(KB // L):
            sl = pl.ds(off + g * L, L)
            sg = srcv[sl]
            dg = dstv[sl]
            ex = _leaky_exp(plsc.load_gather(s_tab, [sg])
                            + plsc.load_gather(t_tab, [dg]) + egv[sl])
            exv[pl.ds(g * L, L)] = ex
            plsc.addupdate_scatter(den_tab, [dg], ex)
        bsl = pl.ds(off, KB)
        pltpu.async_copy(h_sh.at[srcv.at[bsl]], rows, sem).wait()

        @plsc.parallel_loop(0, KB, unroll=8)
        def _scale(e):
            exs = plsc.load_gather(exv, [jnp.full((L,), e, _i32)])
            rows[e, :] = rows[e, :] * exs

        pltpu.async_copy(rows, acc_sh.at[dstv.at[bsl]], sem, add=True).wait()

    plsc.subcore_barrier()
    pltpu.sync_copy(acc_sh.at[rows_sl, :], acc_out.at[c, rows_sl, :])
    pltpu.sync_copy(den_tab, den_out.at[c, sid])


@functools.partial(
    pl.kernel,
    out_type=[
        jax.ShapeDtypeStruct((NC, NP, 64), _f32),
        jax.ShapeDtypeStruct((NS, NP), _f32),
    ],
    mesh=_MESH,
    scratch_types=[
        pltpu.VMEM((NP,), _f32),
        pltpu.VMEM((NP,), _f32),
        pltpu.VMEM((NP,), _f32),
        pltpu.VMEM((CH,), _i32),
        pltpu.VMEM((CH,), _i32),
        pltpu.VMEM((CH,), _f32),
        pltpu.VMEM((KB,), _f32),
        pltpu.VMEM((KB, 64), _f32),
        pltpu.VMEM_SHARED((NP, 64), _f32),
        pltpu.VMEM_SHARED((NP, 64), _f32),
        pltpu.SemaphoreType.DMA,
    ],
    compiler_params=_SC_PARAMS,
)
def _sc_layer128(src_hbm, dst_hbm, eg_hbm, s_hbm, t_hbm, h_hbm, z64_hbm,
                 z1_hbm, acc_out, den_out,
                 s_tab, t_tab, den_tab, srcv, dstv, egv, exv, rows,
                 h_sh, acc_sh, sem):
    c = lax.axis_index("c")
    sid = lax.axis_index("s")
    rows_sl = pl.ds(sid * NPT, NPT)
    col0 = c * 64

    base = sid * ET
    pltpu.sync_copy(s_hbm, s_tab)
    pltpu.sync_copy(t_hbm, t_tab)
    pltpu.sync_copy(z1_hbm, den_tab)
    pltpu.sync_copy(h_hbm.at[rows_sl, pl.ds(col0, 64)], h_sh.at[rows_sl, :])
    pltpu.sync_copy(z64_hbm.at[rows_sl, :], acc_sh.at[rows_sl, :])
    plsc.subcore_barrier()

    @pl.loop(0, NCH)
    def _chunk(ci):
        coff = base + ci * CH
        pltpu.sync_copy(src_hbm.at[pl.ds(coff, CH)], srcv)
        pltpu.sync_copy(dst_hbm.at[pl.ds(coff, CH)], dstv)
        pltpu.sync_copy(eg_hbm.at[pl.ds(coff, CH)], egv)

        @pl.loop(0, NBC)
        def _batch(b):
            off = b * KB
            for g in range(KB // L):
                sl = pl.ds(off + g * L, L)
                sg = srcv[sl]
                dg = dstv[sl]
                ex = _leaky_exp(plsc.load_gather(s_tab, [sg])
                                + plsc.load_gather(t_tab, [dg]) + egv[sl])
                exv[pl.ds(g * L, L)] = ex

                @pl.when(c == 0)
                def _():
                    plsc.addupdate_scatter(den_tab, [dg], ex)

            bsl = pl.ds(off, KB)
            pltpu.async_copy(h_sh.at[srcv.at[bsl]], rows, sem).wait()

            @plsc.parallel_loop(0, KB, unroll=4)
            def _scale(e):
                exs = plsc.load_gather(exv, [jnp.full((L,), e, _i32)])
                for k in range(64 // L):
                    sl = pl.ds(k * L, L)
                    rows[e, sl] = rows[e, sl] * exs

            pltpu.async_copy(rows, acc_sh.at[dstv.at[bsl]], sem,
                             add=True).wait()

    plsc.subcore_barrier()
    pltpu.sync_copy(acc_sh.at[rows_sl, :], acc_out.at[c, rows_sl, :])

    @pl.when(c == 0)
    def _den_epilogue():
        pltpu.sync_copy(den_tab, den_out.at[sid])


def _tc_pre_body(x_ref, w1_ref, asrc_ref, adst_ref, h1_ref, s1_ref, t1_ref):
    h1 = jnp.dot(x_ref[...], w1_ref[...], preferred_element_type=_f32)
    h1p = jnp.concatenate([h1, jnp.zeros((NP - N, 16), _f32)], axis=0)
    h1_ref[...] = h1p
    s1_ref[...] = h1p @ asrc_ref[...]
    t1_ref[...] = h1p @ adst_ref[...]


_tc_pre = pl.pallas_call(
    _tc_pre_body,
    out_shape=[
        jax.ShapeDtypeStruct((NP, 16), _f32),
        jax.ShapeDtypeStruct((NP,), _f32),
        jax.ShapeDtypeStruct((NP,), _f32),
    ],
)

_EGB = 2560
_NEB = E // _EGB
_NEBP = EP // _EGB


def _tc_edges_body(ei_ref, ea_ref, we1_ref, a1e_ref, we2_ref, a2e_ref,
                   we3_ref, a3e_ref,
                   src_ref, dst_ref, eg1_ref, eg2_ref, eg3_ref):
    i = pl.program_id(0)
    ve1 = we1_ref[...] @ a1e_ref[...]
    ve2 = we2_ref[...] @ a2e_ref[...]
    ve3 = we3_ref[...] @ a3e_ref[...]
    ea = ea_ref[...]
    eg1_ref[...] = (ea @ ve1).reshape(1, 1, _EGB)
    eg2_ref[...] = (ea @ ve2).reshape(1, 1, _EGB)
    eg3_ref[...] = (ea @ ve3).reshape(1, 1, _EGB)
    st = pl.multiple_of(jnp.minimum(i, _NEB - 1) * _EGB, _EGB)
    src = ei_ref[0, pl.ds(st, _EGB)]
    dst = ei_ref[1, pl.ds(st, _EGB)]
    pad = jnp.full((_EGB,), N, _i32)
    real = i < _NEB
    src_ref[...] = jnp.where(real, src, pad).reshape(1, 1, _EGB)
    dst_ref[...] = jnp.where(real, dst, pad).reshape(1, 1, _EGB)


_tc_edges = pl.pallas_call(
    _tc_edges_body,
    grid=(_NEBP,),
    in_specs=[
        pl.BlockSpec((2, E), lambda i: (0, 0)),
        pl.BlockSpec((_EGB, D_EDGE), lambda i: (jnp.minimum(i, _NEB - 1), 0)),
        pl.BlockSpec((D_EDGE, D_EDGE), lambda i: (0, 0)),
        pl.BlockSpec((D_EDGE,), lambda i: (0,)),
        pl.BlockSpec((D_EDGE, D_MID), lambda i: (0, 0)),
        pl.BlockSpec((D_MID,), lambda i: (0,)),
        pl.BlockSpec((D_EDGE, 2), lambda i: (0, 0)),
        pl.BlockSpec((2,), lambda i: (0,)),
    ],
    out_specs=[
        pl.BlockSpec((1, 1, _EGB), lambda i: (i, 0, 0)),
        pl.BlockSpec((1, 1, _EGB), lambda i: (i, 0, 0)),
        pl.BlockSpec((1, 1, _EGB), lambda i: (i, 0, 0)),
        pl.BlockSpec((1, 1, _EGB), lambda i: (i, 0, 0)),
        pl.BlockSpec((1, 1, _EGB), lambda i: (i, 0, 0)),
    ],
    out_shape=[
        jax.ShapeDtypeStruct((_NEBP, 1, _EGB), _i32),
        jax.ShapeDtypeStruct((_NEBP, 1, _EGB), _i32),
        jax.ShapeDtypeStruct((_NEBP, 1, _EGB), _f32),
        jax.ShapeDtypeStruct((_NEBP, 1, _EGB), _f32),
        jax.ShapeDtypeStruct((_NEBP, 1, _EGB), _f32),
    ],
)


def _tc_post1_body(acc_ref, den_ref, b1_ref, w2_ref, asrc_ref, adst_ref,
                   h2_ref, s2_ref, t2_ref):
    d = jnp.sum(den_ref[...], axis=0)
    inv = 1.0 / (d + 1e-16)
    out1 = (acc_ref[0] + acc_ref[1]) * inv[:, None] + b1_ref[...][None, :]
    x2 = jnp.maximum(out1, 0.0)
    h2 = jnp.dot(x2, w2_ref[...], preferred_element_type=_f32)
    h2_ref[...] = h2
    s2_ref[...] = h2 @ asrc_ref[...]
    t2_ref[...] = h2 @ adst_ref[...]


_tc_post1 = pl.pallas_call(
    _tc_post1_body,
    out_shape=[
        jax.ShapeDtypeStruct((NP, D_MID), _f32),
        jax.ShapeDtypeStruct((NP,), _f32),
        jax.ShapeDtypeStruct((NP,), _f32),
    ],
)


def _tc_post2_body(acc_ref, den_ref, b2_ref, w3_ref, asrc_ref, adst_ref,
                   h3_ref, s3_ref, t3_ref):
    d = jnp.sum(den_ref[...], axis=0)
    inv = 1.0 / (d + 1e-16)
    out2 = jnp.concatenate([acc_ref[0], acc_ref[1]], axis=1)
    out2 = out2 * inv[:, None] + b2_ref[...][None, :]
    w3p = jnp.concatenate([w3_ref[...], jnp.zeros((D_MID, 14), _f32)], axis=1)
    a3s = jnp.concatenate([asrc_ref[...], jnp.zeros((14,), _f32)])
    a3d = jnp.concatenate([adst_ref[...], jnp.zeros((14,), _f32)])
    h3 = jnp.dot(out2, w3p, preferred_element_type=_f32)
    h3_ref[...] = h3
    s3_ref[...] = h3 @ a3s
    t3_ref[...] = h3 @ a3d


_tc_post2 = pl.pallas_call(
    _tc_post2_body,
    out_shape=[
        jax.ShapeDtypeStruct((NP, 16), _f32),
        jax.ShapeDtypeStruct((NP,), _f32),
        jax.ShapeDtypeStruct((NP,), _f32),
    ],
)


def _tc_post3_body(acc_ref, den_ref, b3_ref, wc_ref, bc_ref, out_ref):
    d = jnp.sum(den_ref[...], axis=0)
    inv = 1.0 / (d + 1e-16)
    out3 = (acc_ref[0] + acc_ref[1])[:, :2] * inv[:, None] + b3_ref[...][None, :]
    logits = jnp.dot(out3, wc_ref[...], preferred_element_type=_f32)
    logits = logits + bc_ref[...][None, :]
    row = lax.broadcasted_iota(_i32, (NP, 1), 0)
    valid = row < N
    neg = jnp.float32(-jnp.inf)
    masked = jnp.where(valid, logits, neg)
    mx = jnp.max(masked, axis=0, keepdims=True)
    z = jnp.where(valid, jnp.exp(logits - mx), 0.0)
    lse = jnp.log(jnp.sum(z, axis=0, keepdims=True)) + mx
    out_ref[...] = lax.slice(logits - lse, (0, 0), (N, 21))


_tc_post3 = pl.pallas_call(
    _tc_post3_body,
    out_shape=jax.ShapeDtypeStruct((N, 21), _f32),
)


def kernel(x, edge_index, edge_attr, W1, We1, a1_src, a1_dst, a1_edge, b1,
           W2, We2, a2_src, a2_dst, a2_edge, b2,
           W3, We3, a3_src, a3_dst, a3_edge, b3, Wc, bc):
    z1 = jnp.zeros((NP,), _f32)
    z16 = jnp.zeros((NP, 16), _f32)
    z64 = jnp.zeros((NP, 64), _f32)

    h1, s1, t1 = _tc_pre(x, W1, a1_src, a1_dst)
    src, dst, eg1, eg2, eg3 = [
        a.reshape(EP) for a in _tc_edges(
            edge_index, edge_attr, We1, a1_edge, We2, a2_edge, We3, a3_edge)]

    acc1, den1 = _sc_layer16(src, dst, eg1, s1, t1, h1, z16, z1)
    h2, s2, t2 = _tc_post1(acc1, den1.reshape(NW, NP), b1, W2, a2_src, a2_dst)

    acc2, den2 = _sc_layer128(src, dst, eg2, s2, t2, h2, z64, z1)
    h3, s3, t3 = _tc_post2(acc2, den2, b2, W3, a3_src, a3_dst)

    acc3, den3 = _sc_layer16(src, dst, eg3, s3, t3, h3, z16, z1)
    return _tc_post3(acc3, den3.reshape(NW, NP), b3, Wc, bc)


# speedup vs baseline: 39.1251x; 1.3582x over previous
import functools

import jax
import jax.numpy as jnp
from jax import lax
from jax.experimental import pallas as pl
from jax.experimental.pallas import tpu as pltpu
from jax.experimental.pallas import tpu_sc as plsc

N = 10000
E = 320000
D_IN = 128
D_EDGE = 16
D_MID = 128

NC = 2
NS = 16
L = 16
NW = NC * NS

NP = 10240
NPT = NP // NS
EP = 327680
KB = 128
EW = EP // NW
NB = EW // KB
ET = EP // NS
CH = 2048
NCH = ET // CH
NBC = CH // KB

_f32 = jnp.float32
_i32 = jnp.int32

_MESH = plsc.VectorSubcoreMesh(core_axis_name="c", subcore_axis_name="s")
_SC_PARAMS = pltpu.CompilerParams(
    needs_layout_passes=False, use_tc_tiling_on_sc=False)


def _leaky_exp(lg):
    lg = jnp.where(lg >= 0, lg, 0.2 * lg)
    return jnp.exp(lg)


@functools.partial(
    pl.kernel,
    out_type=[
        jax.ShapeDtypeStruct((NC, NP, 16), _f32),
        jax.ShapeDtypeStruct((NC, NS, NP), _f32),
    ],
    mesh=_MESH,
    scratch_types=[
        pltpu.VMEM((NP,), _f32),
        pltpu.VMEM((NP,), _f32),
        pltpu.VMEM((NP,), _f32),
        pltpu.VMEM((EW,), _i32),
        pltpu.VMEM((EW,), _i32),
        pltpu.VMEM((EW,), _f32),
        pltpu.VMEM((KB,), _f32),
        pltpu.VMEM((KB, 16), _f32),
        pltpu.VMEM_SHARED((NP, 16), _f32),
        pltpu.VMEM_SHARED((NP, 16), _f32),
        pltpu.SemaphoreType.DMA,
    ],
    compiler_params=_SC_PARAMS,
)
def _sc_layer16(src_hbm, dst_hbm, eg_hbm, s_hbm, t_hbm, h_hbm, z16_hbm,
                z1_hbm, acc_out, den_out,
                s_tab, t_tab, den_tab, srcv, dstv, egv, exv, rows,
                h_sh, acc_sh, sem):
    c = lax.axis_index("c")
    sid = lax.axis_index("s")
    wid = c * NS + sid
    rows_sl = pl.ds(sid * NPT, NPT)

    base = wid * EW
    pltpu.sync_copy(s_hbm, s_tab)
    pltpu.sync_copy(t_hbm, t_tab)
    pltpu.sync_copy(z1_hbm, den_tab)
    pltpu.sync_copy(src_hbm.at[pl.ds(base, EW)], srcv)
    pltpu.sync_copy(dst_hbm.at[pl.ds(base, EW)], dstv)
    pltpu.sync_copy(eg_hbm.at[pl.ds(base, EW)], egv)
    pltpu.sync_copy(h_hbm.at[rows_sl, :], h_sh.at[rows_sl, :])
    pltpu.sync_copy(z16_hbm.at[rows_sl, :], acc_sh.at[rows_sl, :])
    plsc.subcore_barrier()

    @pl.loop(0, NB)
    def _batch(b):
        off = b * KB
        for g in range(KB // L):
            sl = pl.ds(off + g * L, L)
            sg = srcv[sl]
            dg = dstv[sl]
            ex = _leaky_exp(plsc.load_gather(s_tab, [sg])
                            + plsc.load_gather(t_tab, [dg]) + egv[sl])
            exv[pl.ds(g * L, L)] = ex
            plsc.addupdate_scatter(den_tab, [dg], ex)
        bsl = pl.ds(off, KB)
        pltpu.async_copy(h_sh.at[srcv.at[bsl]], rows, sem).wait()

        @plsc.parallel_loop(0, KB, unroll=8)
        def _scale(e):
            exs = plsc.load_gather(exv, [jnp.full((L,), e, _i32)])
            rows[e, :] = rows[e, :] * exs

        pltpu.async_copy(rows, acc_sh.at[dstv.at[bsl]], sem, add=True).wait()

    plsc.subcore_barrier()
    pltpu.sync_copy(acc_sh.at[rows_sl, :], acc_out.at[c, rows_sl, :])
    pltpu.sync_copy(den_tab, den_out.at[c, sid])


@functools.partial(
    pl.kernel,
    out_type=[
        jax.ShapeDtypeStruct((NC, NP, 64), _f32),
        jax.ShapeDtypeStruct((NS, NP), _f32),
    ],
    mesh=_MESH,
    scratch_types=[
        pltpu.VMEM((NP,), _f32),
        pltpu.VMEM((NP,), _f32),
        pltpu.VMEM((NP,), _f32),
        pltpu.VMEM((CH,), _i32),
        pltpu.VMEM((CH,), _i32),
        pltpu.VMEM((CH,), _f32),
        pltpu.VMEM((KB,), _f32),
        pltpu.VMEM((KB, 64), _f32),
        pltpu.VMEM_SHARED((NP, 64), _f32),
        pltpu.VMEM_SHARED((NP, 64), _f32),
        pltpu.SemaphoreType.DMA,
    ],
    compiler_params=_SC_PARAMS,
)
def _sc_layer128(src_hbm, dst_hbm, eg_hbm, s_hbm, t_hbm, h_hbm, z64_hbm,
                 z1_hbm, acc_out, den_out,
                 s_tab, t_tab, den_tab, srcv, dstv, egv, exv, rows,
                 h_sh, acc_sh, sem):
    c = lax.axis_index("c")
    sid = lax.axis_index("s")
    rows_sl = pl.ds(sid * NPT, NPT)
    col0 = c * 64

    base = sid * ET
    pltpu.sync_copy(s_hbm, s_tab)
    pltpu.sync_copy(t_hbm, t_tab)
    pltpu.sync_copy(z1_hbm, den_tab)
    pltpu.sync_copy(h_hbm.at[rows_sl, pl.ds(col0, 64)], h_sh.at[rows_sl, :])
    pltpu.sync_copy(z64_hbm.at[rows_sl, :], acc_sh.at[rows_sl, :])
    plsc.subcore_barrier()

    @pl.loop(0, NCH)
    def _chunk(ci):
        coff = base + ci * CH
        pltpu.sync_copy(src_hbm.at[pl.ds(coff, CH)], srcv)
        pltpu.sync_copy(dst_hbm.at[pl.ds(coff, CH)], dstv)
        pltpu.sync_copy(eg_hbm.at[pl.ds(coff, CH)], egv)

        @pl.loop(0, NBC)
        def _batch(b):
            off = b * KB
            for g in range(KB // L):
                sl = pl.ds(off + g * L, L)
                sg = srcv[sl]
                dg = dstv[sl]
                ex = _leaky_exp(plsc.load_gather(s_tab, [sg])
                                + plsc.load_gather(t_tab, [dg]) + egv[sl])
                exv[pl.ds(g * L, L)] = ex

                @pl.when(c == 0)
                def _():
                    plsc.addupdate_scatter(den_tab, [dg], ex)

            bsl = pl.ds(off, KB)
            pltpu.async_copy(h_sh.at[srcv.at[bsl]], rows, sem).wait()

            @plsc.parallel_loop(0, KB, unroll=4)
            def _scale(e):
                exs = plsc.load_gather(exv, [jnp.full((L,), e, _i32)])
                for k in range(64 // L):
                    sl = pl.ds(k * L, L)
                    rows[e, sl] = rows[e, sl] * exs

            pltpu.async_copy(rows, acc_sh.at[dstv.at[bsl]], sem,
                             add=True).wait()

    plsc.subcore_barrier()
    pltpu.sync_copy(acc_sh.at[rows_sl, :], acc_out.at[c, rows_sl, :])

    @pl.when(c == 0)
    def _den_epilogue():
        pltpu.sync_copy(den_tab, den_out.at[sid])


def _tc_pre_body(x_ref, w1_ref, asrc_ref, adst_ref, h1_ref, s1_ref, t1_ref):
    h1 = jnp.dot(x_ref[...], w1_ref[...], preferred_element_type=_f32)
    h1p = jnp.concatenate([h1, jnp.zeros((NP - N, 16), _f32)], axis=0)
    h1_ref[...] = h1p
    s1_ref[...] = h1p @ asrc_ref[...]
    t1_ref[...] = h1p @ adst_ref[...]


_tc_pre = pl.pallas_call(
    _tc_pre_body,
    out_shape=[
        jax.ShapeDtypeStruct((NP, 16), _f32),
        jax.ShapeDtypeStruct((NP,), _f32),
        jax.ShapeDtypeStruct((NP,), _f32),
    ],
)

_EGB = 20480
_NEG = EP // _EGB


def _tc_eg_body(ea_ref, we1_ref, a1e_ref, we2_ref, a2e_ref,
                we3_ref, a3e_ref, eg1_ref, eg2_ref, eg3_ref):
    ve1 = we1_ref[...] @ a1e_ref[...]
    ve2 = we2_ref[...] @ a2e_ref[...]
    ve3 = we3_ref[...] @ a3e_ref[...]
    ea = ea_ref[...]
    eg1_ref[...] = (ea @ ve1).reshape(1, 8, _EGB // 8)
    eg2_ref[...] = (ea @ ve2).reshape(1, 8, _EGB // 8)
    eg3_ref[...] = (ea @ ve3).reshape(1, 8, _EGB // 8)


_tc_eg = pl.pallas_call(
    _tc_eg_body,
    grid=(_NEG,),
    in_specs=[
        pl.BlockSpec((_EGB, D_EDGE), lambda i: (i, 0)),
        pl.BlockSpec((D_EDGE, D_EDGE), lambda i: (0, 0)),
        pl.BlockSpec((D_EDGE,), lambda i: (0,)),
        pl.BlockSpec((D_EDGE, D_MID), lambda i: (0, 0)),
        pl.BlockSpec((D_MID,), lambda i: (0,)),
        pl.BlockSpec((D_EDGE, 2), lambda i: (0, 0)),
        pl.BlockSpec((2,), lambda i: (0,)),
    ],
    out_specs=[
        pl.BlockSpec((1, 8, _EGB // 8), lambda i: (i, 0, 0)),
        pl.BlockSpec((1, 8, _EGB // 8), lambda i: (i, 0, 0)),
        pl.BlockSpec((1, 8, _EGB // 8), lambda i: (i, 0, 0)),
    ],
    out_shape=[
        jax.ShapeDtypeStruct((_NEG, 8, _EGB // 8), _f32),
        jax.ShapeDtypeStruct((_NEG, 8, _EGB // 8), _f32),
        jax.ShapeDtypeStruct((_NEG, 8, _EGB // 8), _f32),
    ],
)


def _tc_post1_body(acc_ref, den_ref, b1_ref, w2_ref, asrc_ref, adst_ref,
                   h2_ref, s2_ref, t2_ref):
    d = jnp.sum(den_ref[...], axis=0)
    inv = 1.0 / (d + 1e-16)
    out1 = (acc_ref[0] + acc_ref[1]) * inv[:, None] + b1_ref[...][None, :]
    x2 = jnp.maximum(out1, 0.0)
    h2 = jnp.dot(x2, w2_ref[...], preferred_element_type=_f32)
    h2_ref[...] = h2
    s2_ref[...] = h2 @ asrc_ref[...]
    t2_ref[...] = h2 @ adst_ref[...]


_tc_post1 = pl.pallas_call(
    _tc_post1_body,
    out_shape=[
        jax.ShapeDtypeStruct((NP, D_MID), _f32),
        jax.ShapeDtypeStruct((NP,), _f32),
        jax.ShapeDtypeStruct((NP,), _f32),
    ],
)


def _tc_post2_body(acc_ref, den_ref, b2_ref, w3_ref, asrc_ref, adst_ref,
                   h3_ref, s3_ref, t3_ref):
    d = jnp.sum(den_ref[...], axis=0)
    inv = 1.0 / (d + 1e-16)
    out2 = jnp.concatenate([acc_ref[0], acc_ref[1]], axis=1)
    out2 = out2 * inv[:, None] + b2_ref[...][None, :]
    w3p = jnp.concatenate([w3_ref[...], jnp.zeros((D_MID, 14), _f32)], axis=1)
    a3s = jnp.concatenate([asrc_ref[...], jnp.zeros((14,), _f32)])
    a3d = jnp.concatenate([adst_ref[...], jnp.zeros((14,), _f32)])
    h3 = jnp.dot(out2, w3p, preferred_element_type=_f32)
    h3_ref[...] = h3
    s3_ref[...] = h3 @ a3s
    t3_ref[...] = h3 @ a3d


_tc_post2 = pl.pallas_call(
    _tc_post2_body,
    out_shape=[
        jax.ShapeDtypeStruct((NP, 16), _f32),
        jax.ShapeDtypeStruct((NP,), _f32),
        jax.ShapeDtypeStruct((NP,), _f32),
    ],
)


def _tc_post3_body(acc_ref, den_ref, b3_ref, wc_ref, bc_ref, out_ref):
    d = jnp.sum(den_ref[...], axis=0)
    inv = 1.0 / (d + 1e-16)
    out3 = (acc_ref[0] + acc_ref[1])[:, :2] * inv[:, None] + b3_ref[...][None, :]
    logits = jnp.dot(out3, wc_ref[...], preferred_element_type=_f32)
    logits = logits + bc_ref[...][None, :]
    row = lax.broadcasted_iota(_i32, (NP, 1), 0)
    valid = row < N
    neg = jnp.float32(-jnp.inf)
    masked = jnp.where(valid, logits, neg)
    mx = jnp.max(masked, axis=0, keepdims=True)
    z = jnp.where(valid, jnp.exp(logits - mx), 0.0)
    lse = jnp.log(jnp.sum(z, axis=0, keepdims=True)) + mx
    out_ref[...] = lax.slice(logits - lse, (0, 0), (N, 21))


_tc_post3 = pl.pallas_call(
    _tc_post3_body,
    out_shape=jax.ShapeDtypeStruct((N, 21), _f32),
)


def kernel(x, edge_index, edge_attr, W1, We1, a1_src, a1_dst, a1_edge, b1,
           W2, We2, a2_src, a2_dst, a2_edge, b2,
           W3, We3, a3_src, a3_dst, a3_edge, b3, Wc, bc):
    z1 = jnp.zeros((NP,), _f32)
    z16 = jnp.zeros((NP, 16), _f32)
    z64 = jnp.zeros((NP, 64), _f32)

    h1, s1, t1 = _tc_pre(x, W1, a1_src, a1_dst)
    src = jnp.pad(edge_index[0], (0, EP - E), constant_values=N)
    dst = jnp.pad(edge_index[1], (0, EP - E), constant_values=N)
    eg1, eg2, eg3 = [
        a.reshape(EP) for a in _tc_eg(
            edge_attr, We1, a1_edge, We2, a2_edge, We3, a3_edge)]

    acc1, den1 = _sc_layer16(src, dst, eg1, s1, t1, h1, z16, z1)
    h2, s2, t2 = _tc_post1(acc1, den1.reshape(NW, NP), b1, W2, a2_src, a2_dst)

    acc2, den2 = _sc_layer128(src, dst, eg2, s2, t2, h2, z64, z1)
    h3, s3, t3 = _tc_post2(acc2, den2, b2, W3, a3_src, a3_dst)

    acc3, den3 = _sc_layer16(src, dst, eg3, s3, t3, h3, z16, z1)
    return _tc_post3(acc3, den3.reshape(NW, NP), b3, Wc, bc)


# speedup vs baseline: 43.2983x; 1.1067x over previous
import functools

import jax
import jax.numpy as jnp
from jax import lax
from jax.experimental import pallas as pl
from jax.experimental.pallas import tpu as pltpu
from jax.experimental.pallas import tpu_sc as plsc

N = 10000
E = 320000
D_IN = 128
D_EDGE = 16
D_MID = 128

NC = 2
NS = 16
L = 16
NW = NC * NS

NP = 10240
NPT = NP // NS
EP = 327680
KB = 128
EW = EP // NW
NB = EW // KB
ET = EP // NS
CH = 2048
NCH = ET // CH
NBC = CH // KB

_f32 = jnp.float32
_i32 = jnp.int32

_MESH = plsc.VectorSubcoreMesh(core_axis_name="c", subcore_axis_name="s")
_SC_PARAMS = pltpu.CompilerParams(
    needs_layout_passes=False, use_tc_tiling_on_sc=False)


def _leaky_exp(lg):
    lg = jnp.where(lg >= 0, lg, 0.2 * lg)
    return jnp.exp(lg)


@functools.partial(
    pl.kernel,
    out_type=[
        jax.ShapeDtypeStruct((NC, NP, 16), _f32),
        jax.ShapeDtypeStruct((NC, NS, NP), _f32),
    ],
    mesh=_MESH,
    scratch_types=[
        pltpu.VMEM((NP,), _f32),
        pltpu.VMEM((NP,), _f32),
        pltpu.VMEM((NP,), _f32),
        pltpu.VMEM((EW,), _i32),
        pltpu.VMEM((EW,), _i32),
        pltpu.VMEM((EW,), _f32),
        pltpu.VMEM((KB,), _f32),
        pltpu.VMEM((KB, 16), _f32),
        pltpu.VMEM_SHARED((NP, 16), _f32),
        pltpu.VMEM_SHARED((NP, 16), _f32),
        pltpu.SemaphoreType.DMA,
    ],
    compiler_params=_SC_PARAMS,
)
def _sc_layer16(src_hbm, dst_hbm, eg_hbm, s_hbm, t_hbm, h_hbm, z16_hbm,
                z1_hbm, acc_out, den_out,
                s_tab, t_tab, den_tab, srcv, dstv, egv, exv, rows,
                h_sh, acc_sh, sem):
    c = lax.axis_index("c")
    sid = lax.axis_index("s")
    wid = c * NS + sid
    rows_sl = pl.ds(sid * NPT, NPT)

    base = wid * EW
    pltpu.sync_copy(s_hbm, s_tab)
    pltpu.sync_copy(t_hbm, t_tab)
    pltpu.sync_copy(z1_hbm, den_tab)
    pltpu.sync_copy(src_hbm.at[pl.ds(base, EW)], srcv)
    pltpu.sync_copy(dst_hbm.at[pl.ds(base, EW)], dstv)
    pltpu.sync_copy(eg_hbm.at[pl.ds(base, EW)], egv)
    pltpu.sync_copy(h_hbm.at[rows_sl, :], h_sh.at[rows_sl, :])
    pltpu.sync_copy(z16_hbm.at[rows_sl, :], acc_sh.at[rows_sl, :])
    plsc.subcore_barrier()

    @pl.loop(0, NB)
    def _batch(b):
        off = b * KB
        bsl = pl.ds(off, KB)
        cp = pltpu.async_copy(h_sh.at[srcv.at[bsl]], rows, sem)
        for g in range(KB // L):
            sl = pl.ds(off + g * L, L)
            sg = srcv[sl]
            dg = dstv[sl]
            ex = _leaky_exp(plsc.load_gather(s_tab, [sg])
                            + plsc.load_gather(t_tab, [dg]) + egv[sl])
            exv[pl.ds(g * L, L)] = ex
            plsc.addupdate_scatter(den_tab, [dg], ex)
        cp.wait()

        @plsc.parallel_loop(0, KB, unroll=8)
        def _scale(e):
            exs = plsc.load_gather(exv, [jnp.full((L,), e, _i32)])
            rows[e, :] = rows[e, :] * exs

        pltpu.async_copy(rows, acc_sh.at[dstv.at[bsl]], sem, add=True).wait()

    plsc.subcore_barrier()
    pltpu.sync_copy(acc_sh.at[rows_sl, :], acc_out.at[c, rows_sl, :])
    pltpu.sync_copy(den_tab, den_out.at[c, sid])


@functools.partial(
    pl.kernel,
    out_type=[
        jax.ShapeDtypeStruct((NC, NP, 64), _f32),
        jax.ShapeDtypeStruct((NS, NP), _f32),
    ],
    mesh=_MESH,
    scratch_types=[
        pltpu.VMEM((NP,), _f32),
        pltpu.VMEM((NP,), _f32),
        pltpu.VMEM((NP,), _f32),
        pltpu.VMEM((CH,), _i32),
        pltpu.VMEM((CH,), _i32),
        pltpu.VMEM((CH,), _f32),
        pltpu.VMEM((KB,), _f32),
        pltpu.VMEM((KB, 64), _f32),
        pltpu.VMEM_SHARED((NP, 64), _f32),
        pltpu.VMEM_SHARED((NP, 64), _f32),
        pltpu.SemaphoreType.DMA,
    ],
    compiler_params=_SC_PARAMS,
)
def _sc_layer128(src_hbm, dst_hbm, eg_hbm, s_hbm, t_hbm, h_hbm, z64_hbm,
                 z1_hbm, acc_out, den_out,
                 s_tab, t_tab, den_tab, srcv, dstv, egv, exv, rows,
                 h_sh, acc_sh, sem):
    c = lax.axis_index("c")
    sid = lax.axis_index("s")
    rows_sl = pl.ds(sid * NPT, NPT)
    col0 = c * 64

    base = sid * ET
    pltpu.sync_copy(s_hbm, s_tab)
    pltpu.sync_copy(t_hbm, t_tab)
    pltpu.sync_copy(z1_hbm, den_tab)
    pltpu.sync_copy(h_hbm.at[rows_sl, pl.ds(col0, 64)], h_sh.at[rows_sl, :])
    pltpu.sync_copy(z64_hbm.at[rows_sl, :], acc_sh.at[rows_sl, :])
    plsc.subcore_barrier()

    @pl.loop(0, NCH)
    def _chunk(ci):
        coff = base + ci * CH
        pltpu.sync_copy(src_hbm.at[pl.ds(coff, CH)], srcv)
        pltpu.sync_copy(dst_hbm.at[pl.ds(coff, CH)], dstv)
        pltpu.sync_copy(eg_hbm.at[pl.ds(coff, CH)], egv)

        @pl.loop(0, NBC)
        def _batch(b):
            off = b * KB
            bsl = pl.ds(off, KB)
            cp = pltpu.async_copy(h_sh.at[srcv.at[bsl]], rows, sem)
            for g in range(KB // L):
                sl = pl.ds(off + g * L, L)
                sg = srcv[sl]
                dg = dstv[sl]
                ex = _leaky_exp(plsc.load_gather(s_tab, [sg])
                                + plsc.load_gather(t_tab, [dg]) + egv[sl])
                exv[pl.ds(g * L, L)] = ex

                @pl.when(c == 0)
                def _():
                    plsc.addupdate_scatter(den_tab, [dg], ex)

            cp.wait()

            @plsc.parallel_loop(0, KB, unroll=4)
            def _scale(e):
                exs = plsc.load_gather(exv, [jnp.full((L,), e, _i32)])
                for k in range(64 // L):
                    sl = pl.ds(k * L, L)
                    rows[e, sl] = rows[e, sl] * exs

            pltpu.async_copy(rows, acc_sh.at[dstv.at[bsl]], sem,
                             add=True).wait()

    plsc.subcore_barrier()
    pltpu.sync_copy(acc_sh.at[rows_sl, :], acc_out.at[c, rows_sl, :])

    @pl.when(c == 0)
    def _den_epilogue():
        pltpu.sync_copy(den_tab, den_out.at[sid])


def _tc_pre_body(x_ref, w1_ref, asrc_ref, adst_ref, h1_ref, s1_ref, t1_ref):
    h1 = jnp.dot(x_ref[...], w1_ref[...], preferred_element_type=_f32)
    h1p = jnp.concatenate([h1, jnp.zeros((NP - N, 16), _f32)], axis=0)
    h1_ref[...] = h1p
    s1_ref[...] = h1p @ asrc_ref[...]
    t1_ref[...] = h1p @ adst_ref[...]


_tc_pre = pl.pallas_call(
    _tc_pre_body,
    out_shape=[
        jax.ShapeDtypeStruct((NP, 16), _f32),
        jax.ShapeDtypeStruct((NP,), _f32),
        jax.ShapeDtypeStruct((NP,), _f32),
    ],
)

_EGB = 20480
_NEG = EP // _EGB


def _tc_eg_body(ea_ref, we1_ref, a1e_ref, we2_ref, a2e_ref,
                we3_ref, a3e_ref, eg1_ref, eg2_ref, eg3_ref):
    ve1 = we1_ref[...] @ a1e_ref[...]
    ve2 = we2_ref[...] @ a2e_ref[...]
    ve3 = we3_ref[...] @ a3e_ref[...]
    ea = ea_ref[...]
    eg1_ref[...] = (ea @ ve1).reshape(1, 8, _EGB // 8)
    eg2_ref[...] = (ea @ ve2).reshape(1, 8, _EGB // 8)
    eg3_ref[...] = (ea @ ve3).reshape(1, 8, _EGB // 8)


_tc_eg = pl.pallas_call(
    _tc_eg_body,
    grid=(_NEG,),
    in_specs=[
        pl.BlockSpec((_EGB, D_EDGE), lambda i: (i, 0)),
        pl.BlockSpec((D_EDGE, D_EDGE), lambda i: (0, 0)),
        pl.BlockSpec((D_EDGE,), lambda i: (0,)),
        pl.BlockSpec((D_EDGE, D_MID), lambda i: (0, 0)),
        pl.BlockSpec((D_MID,), lambda i: (0,)),
        pl.BlockSpec((D_EDGE, 2), lambda i: (0, 0)),
        pl.BlockSpec((2,), lambda i: (0,)),
    ],
    out_specs=[
        pl.BlockSpec((1, 8, _EGB // 8), lambda i: (i, 0, 0)),
        pl.BlockSpec((1, 8, _EGB // 8), lambda i: (i, 0, 0)),
        pl.BlockSpec((1, 8, _EGB // 8), lambda i: (i, 0, 0)),
    ],
    out_shape=[
        jax.ShapeDtypeStruct((_NEG, 8, _EGB // 8), _f32),
        jax.ShapeDtypeStruct((_NEG, 8, _EGB // 8), _f32),
        jax.ShapeDtypeStruct((_NEG, 8, _EGB // 8), _f32),
    ],
)


def _tc_post1_body(acc_ref, den_ref, b1_ref, w2_ref, asrc_ref, adst_ref,
                   h2_ref, s2_ref, t2_ref):
    d = jnp.sum(den_ref[...], axis=0)
    inv = 1.0 / (d + 1e-16)
    out1 = (acc_ref[0] + acc_ref[1]) * inv[:, None] + b1_ref[...][None, :]
    x2 = jnp.maximum(out1, 0.0)
    h2 = jnp.dot(x2, w2_ref[...], preferred_element_type=_f32)
    h2_ref[...] = h2
    s2_ref[...] = h2 @ asrc_ref[...]
    t2_ref[...] = h2 @ adst_ref[...]


_tc_post1 = pl.pallas_call(
    _tc_post1_body,
    out_shape=[
        jax.ShapeDtypeStruct((NP, D_MID), _f32),
        jax.ShapeDtypeStruct((NP,), _f32),
        jax.ShapeDtypeStruct((NP,), _f32),
    ],
)


def _tc_post2_body(acc_ref, den_ref, b2_ref, w3_ref, asrc_ref, adst_ref,
                   h3_ref, s3_ref, t3_ref):
    d = jnp.sum(den_ref[...], axis=0)
    inv = 1.0 / (d + 1e-16)
    out2 = jnp.concatenate([acc_ref[0], acc_ref[1]], axis=1)
    out2 = out2 * inv[:, None] + b2_ref[...][None, :]
    w3p = jnp.concatenate([w3_ref[...], jnp.zeros((D_MID, 14), _f32)], axis=1)
    a3s = jnp.concatenate([asrc_ref[...], jnp.zeros((14,), _f32)])
    a3d = jnp.concatenate([adst_ref[...], jnp.zeros((14,), _f32)])
    h3 = jnp.dot(out2, w3p, preferred_element_type=_f32)
    h3_ref[...] = h3
    s3_ref[...] = h3 @ a3s
    t3_ref[...] = h3 @ a3d


_tc_post2 = pl.pallas_call(
    _tc_post2_body,
    out_shape=[
        jax.ShapeDtypeStruct((NP, 16), _f32),
        jax.ShapeDtypeStruct((NP,), _f32),
        jax.ShapeDtypeStruct((NP,), _f32),
    ],
)


def _tc_post3_body(acc_ref, den_ref, b3_ref, wc_ref, bc_ref, out_ref):
    d = jnp.sum(den_ref[...], axis=0)
    inv = 1.0 / (d + 1e-16)
    out3 = (acc_ref[0] + acc_ref[1])[:, :2] * inv[:, None] + b3_ref[...][None, :]
    logits = jnp.dot(out3, wc_ref[...], preferred_element_type=_f32)
    logits = logits + bc_ref[...][None, :]
    row = lax.broadcasted_iota(_i32, (NP, 1), 0)
    valid = row < N
    neg = jnp.float32(-jnp.inf)
    masked = jnp.where(valid, logits, neg)
    mx = jnp.max(masked, axis=0, keepdims=True)
    z = jnp.where(valid, jnp.exp(logits - mx), 0.0)
    lse = jnp.log(jnp.sum(z, axis=0, keepdims=True)) + mx
    out_ref[...] = lax.slice(logits - lse, (0, 0), (N, 21))


_tc_post3 = pl.pallas_call(
    _tc_post3_body,
    out_shape=jax.ShapeDtypeStruct((N, 21), _f32),
)


def kernel(x, edge_index, edge_attr, W1, We1, a1_src, a1_dst, a1_edge, b1,
           W2, We2, a2_src, a2_dst, a2_edge, b2,
           W3, We3, a3_src, a3_dst, a3_edge, b3, Wc, bc):
    z1 = jnp.zeros((NP,), _f32)
    z16 = jnp.zeros((NP, 16), _f32)
    z64 = jnp.zeros((NP, 64), _f32)

    h1, s1, t1 = _tc_pre(x, W1, a1_src, a1_dst)
    src = jnp.pad(edge_index[0], (0, EP - E), constant_values=N)
    dst = jnp.pad(edge_index[1], (0, EP - E), constant_values=N)
    eg1, eg2, eg3 = [
        a.reshape(EP) for a in _tc_eg(
            edge_attr, We1, a1_edge, We2, a2_edge, We3, a3_edge)]

    acc1, den1 = _sc_layer16(src, dst, eg1, s1, t1, h1, z16, z1)
    h2, s2, t2 = _tc_post1(acc1, den1.reshape(NW, NP), b1, W2, a2_src, a2_dst)

    acc2, den2 = _sc_layer128(src, dst, eg2, s2, t2, h2, z64, z1)
    h3, s3, t3 = _tc_post2(acc2, den2, b2, W3, a3_src, a3_dst)

    acc3, den3 = _sc_layer16(src, dst, eg3, s3, t3, h3, z16, z1)
    return _tc_post3(acc3, den3.reshape(NW, NP), b3, Wc, bc)


# speedup vs baseline: 43.7236x; 1.0098x over previous
import functools

import jax
import jax.numpy as jnp
from jax import lax
from jax.experimental import pallas as pl
from jax.experimental.pallas import tpu as pltpu
from jax.experimental.pallas import tpu_sc as plsc

N = 10000
E = 320000
D_IN = 128
D_EDGE = 16
D_MID = 128

NC = 2
NS = 16
L = 16
NW = NC * NS

NP = 10240
NPT = NP // NS
EP = 327680
KB = 128
EW = EP // NW
NB = EW // KB
ET = EP // NS
CH = 2560
NCH = ET // CH
NBC = CH // KB

_f32 = jnp.float32
_i32 = jnp.int32

_MESH = plsc.VectorSubcoreMesh(core_axis_name="c", subcore_axis_name="s")
_SC_PARAMS = pltpu.CompilerParams(
    needs_layout_passes=False, use_tc_tiling_on_sc=False)


def _leaky_exp(lg):
    lg = jnp.where(lg >= 0, lg, 0.2 * lg)
    return jnp.exp(lg)


@functools.partial(
    pl.kernel,
    out_type=[
        jax.ShapeDtypeStruct((NC, NP, 16), _f32),
        jax.ShapeDtypeStruct((NC, NS, NP), _f32),
    ],
    mesh=_MESH,
    scratch_types=[
        pltpu.VMEM((NP,), _f32),
        pltpu.VMEM((NP,), _f32),
        pltpu.VMEM((NP,), _f32),
        pltpu.VMEM((EW,), _i32),
        pltpu.VMEM((EW,), _i32),
        pltpu.VMEM((EW,), _f32),
        pltpu.VMEM((KB,), _f32),
        pltpu.VMEM((KB, 16), _f32),
        pltpu.VMEM_SHARED((NP, 16), _f32),
        pltpu.VMEM_SHARED((NP, 16), _f32),
        pltpu.SemaphoreType.DMA,
    ],
    compiler_params=_SC_PARAMS,
)
def _sc_layer16(src_hbm, dst_hbm, eg_hbm, s_hbm, t_hbm, h_hbm, z16_hbm,
                z1_hbm, acc_out, den_out,
                s_tab, t_tab, den_tab, srcv, dstv, egv, exv, rows,
                h_sh, acc_sh, sem):
    c = lax.axis_index("c")
    sid = lax.axis_index("s")
    wid = c * NS + sid
    rows_sl = pl.ds(sid * NPT, NPT)

    base = wid * EW
    pltpu.sync_copy(s_hbm, s_tab)
    pltpu.sync_copy(t_hbm, t_tab)
    pltpu.sync_copy(z1_hbm, den_tab)
    pltpu.sync_copy(src_hbm.at[pl.ds(base, EW)], srcv)
    pltpu.sync_copy(dst_hbm.at[pl.ds(base, EW)], dstv)
    pltpu.sync_copy(eg_hbm.at[pl.ds(base, EW)], egv)
    pltpu.sync_copy(h_hbm.at[rows_sl, :], h_sh.at[rows_sl, :])
    pltpu.sync_copy(z16_hbm.at[rows_sl, :], acc_sh.at[rows_sl, :])
    plsc.subcore_barrier()

    @pl.loop(0, NB)
    def _batch(b):
        off = b * KB
        bsl = pl.ds(off, KB)
        cp = pltpu.async_copy(h_sh.at[srcv.at[bsl]], rows, sem)
        for g in range(KB // L):
            sl = pl.ds(off + g * L, L)
            sg = srcv[sl]
            dg = dstv[sl]
            ex = _leaky_exp(plsc.load_gather(s_tab, [sg])
                            + plsc.load_gather(t_tab, [dg]) + egv[sl])
            exv[pl.ds(g * L, L)] = ex
            plsc.addupdate_scatter(den_tab, [dg], ex)
        cp.wait()

        @plsc.parallel_loop(0, KB, unroll=8)
        def _scale(e):
            exs = plsc.load_gather(exv, [jnp.full((L,), e, _i32)])
            rows[e, :] = rows[e, :] * exs

        pltpu.async_copy(rows, acc_sh.at[dstv.at[bsl]], sem, add=True).wait()

    plsc.subcore_barrier()
    pltpu.sync_copy(acc_sh.at[rows_sl, :], acc_out.at[c, rows_sl, :])
    pltpu.sync_copy(den_tab, den_out.at[c, sid])


@functools.partial(
    pl.kernel,
    out_type=[
        jax.ShapeDtypeStruct((NC, NP, 64), _f32),
        jax.ShapeDtypeStruct((NS, NP), _f32),
    ],
    mesh=_MESH,
    scratch_types=[
        pltpu.VMEM((NP,), _f32),
        pltpu.VMEM((NP,), _f32),
        pltpu.VMEM((NP,), _f32),
        pltpu.VMEM((CH,), _i32),
        pltpu.VMEM((CH,), _i32),
        pltpu.VMEM((CH,), _f32),
        pltpu.VMEM((KB,), _f32),
        pltpu.VMEM((KB, 64), _f32),
        pltpu.VMEM_SHARED((NP, 64), _f32),
        pltpu.VMEM_SHARED((NP, 64), _f32),
        pltpu.SemaphoreType.DMA,
    ],
    compiler_params=_SC_PARAMS,
)
def _sc_layer128(src_hbm, dst_hbm, eg_hbm, s_hbm, t_hbm, h_hbm, z64_hbm,
                 z1_hbm, acc_out, den_out,
                 s_tab, t_tab, den_tab, srcv, dstv, egv, exv, rows,
                 h_sh, acc_sh, sem):
    c = lax.axis_index("c")
    sid = lax.axis_index("s")
    rows_sl = pl.ds(sid * NPT, NPT)
    col0 = c * 64

    base = sid * ET
    pltpu.sync_copy(s_hbm, s_tab)
    pltpu.sync_copy(t_hbm, t_tab)
    pltpu.sync_copy(z1_hbm, den_tab)
    pltpu.sync_copy(h_hbm.at[rows_sl, pl.ds(col0, 64)], h_sh.at[rows_sl, :])
    pltpu.sync_copy(z64_hbm.at[rows_sl, :], acc_sh.at[rows_sl, :])
    plsc.subcore_barrier()

    @pl.loop(0, NCH)
    def _chunk(ci):
        coff = base + ci * CH
        pltpu.sync_copy(src_hbm.at[pl.ds(coff, CH)], srcv)
        pltpu.sync_copy(dst_hbm.at[pl.ds(coff, CH)], dstv)
        pltpu.sync_copy(eg_hbm.at[pl.ds(coff, CH)], egv)

        @pl.loop(0, NBC)
        def _batch(b):
            off = b * KB
            bsl = pl.ds(off, KB)
            cp = pltpu.async_copy(h_sh.at[srcv.at[bsl]], rows, sem)
            for g in range(KB // L):
                sl = pl.ds(off + g * L, L)
                sg = srcv[sl]
                dg = dstv[sl]
                ex = _leaky_exp(plsc.load_gather(s_tab, [sg])
                                + plsc.load_gather(t_tab, [dg]) + egv[sl])
                exv[pl.ds(g * L, L)] = ex

                @pl.when(c == 0)
                def _():
                    plsc.addupdate_scatter(den_tab, [dg], ex)

            cp.wait()

            @plsc.parallel_loop(0, KB, unroll=4)
            def _scale(e):
                exs = plsc.load_gather(exv, [jnp.full((L,), e, _i32)])
                for k in range(64 // L):
                    sl = pl.ds(k * L, L)
                    rows[e, sl] = rows[e, sl] * exs

            pltpu.async_copy(rows, acc_sh.at[dstv.at[bsl]], sem,
                             add=True).wait()

    plsc.subcore_barrier()
    pltpu.sync_copy(acc_sh.at[rows_sl, :], acc_out.at[c, rows_sl, :])

    @pl.when(c == 0)
    def _den_epilogue():
        pltpu.sync_copy(den_tab, den_out.at[sid])


def _tc_pre_body(x_ref, w1_ref, asrc_ref, adst_ref, h1_ref, s1_ref, t1_ref):
    h1 = jnp.dot(x_ref[...], w1_ref[...], preferred_element_type=_f32)
    h1p = jnp.concatenate([h1, jnp.zeros((NP - N, 16), _f32)], axis=0)
    h1_ref[...] = h1p
    s1_ref[...] = h1p @ asrc_ref[...]
    t1_ref[...] = h1p @ adst_ref[...]


_tc_pre = pl.pallas_call(
    _tc_pre_body,
    out_shape=[
        jax.ShapeDtypeStruct((NP, 16), _f32),
        jax.ShapeDtypeStruct((NP,), _f32),
        jax.ShapeDtypeStruct((NP,), _f32),
    ],
)

_EGB = 20480
_NEG = EP // _EGB


def _tc_eg_body(ea_ref, we1_ref, a1e_ref, we2_ref, a2e_ref,
                we3_ref, a3e_ref, eg1_ref, eg2_ref, eg3_ref):
    ve1 = we1_ref[...] @ a1e_ref[...]
    ve2 = we2_ref[...] @ a2e_ref[...]
    ve3 = we3_ref[...] @ a3e_ref[...]
    ea = ea_ref[...]
    eg1_ref[...] = (ea @ ve1).reshape(1, 8, _EGB // 8)
    eg2_ref[...] = (ea @ ve2).reshape(1, 8, _EGB // 8)
    eg3_ref[...] = (ea @ ve3).reshape(1, 8, _EGB // 8)


_tc_eg = pl.pallas_call(
    _tc_eg_body,
    grid=(_NEG,),
    in_specs=[
        pl.BlockSpec((_EGB, D_EDGE), lambda i: (i, 0)),
        pl.BlockSpec((D_EDGE, D_EDGE), lambda i: (0, 0)),
        pl.BlockSpec((D_EDGE,), lambda i: (0,)),
        pl.BlockSpec((D_EDGE, D_MID), lambda i: (0, 0)),
        pl.BlockSpec((D_MID,), lambda i: (0,)),
        pl.BlockSpec((D_EDGE, 2), lambda i: (0, 0)),
        pl.BlockSpec((2,), lambda i: (0,)),
    ],
    out_specs=[
        pl.BlockSpec((1, 8, _EGB // 8), lambda i: (i, 0, 0)),
        pl.BlockSpec((1, 8, _EGB // 8), lambda i: (i, 0, 0)),
        pl.BlockSpec((1, 8, _EGB // 8), lambda i: (i, 0, 0)),
    ],
    out_shape=[
        jax.ShapeDtypeStruct((_NEG, 8, _EGB // 8), _f32),
        jax.ShapeDtypeStruct((_NEG, 8, _EGB // 8), _f32),
        jax.ShapeDtypeStruct((_NEG, 8, _EGB // 8), _f32),
    ],
)


def _tc_post1_body(acc_ref, den_ref, b1_ref, w2_ref, asrc_ref, adst_ref,
                   h2_ref, s2_ref, t2_ref):
    d = jnp.sum(den_ref[...], axis=0)
    inv = 1.0 / (d + 1e-16)
    out1 = (acc_ref[0] + acc_ref[1]) * inv[:, None] + b1_ref[...][None, :]
    x2 = jnp.maximum(out1, 0.0)
    h2 = jnp.dot(x2, w2_ref[...], preferred_element_type=_f32)
    h2_ref[...] = h2
    s2_ref[...] = h2 @ asrc_ref[...]
    t2_ref[...] = h2 @ adst_ref[...]


_tc_post1 = pl.pallas_call(
    _tc_post1_body,
    out_shape=[
        jax.ShapeDtypeStruct((NP, D_MID), _f32),
        jax.ShapeDtypeStruct((NP,), _f32),
        jax.ShapeDtypeStruct((NP,), _f32),
    ],
)


def _tc_post2_body(acc_ref, den_ref, b2_ref, w3_ref, asrc_ref, adst_ref,
                   h3_ref, s3_ref, t3_ref):
    d = jnp.sum(den_ref[...], axis=0)
    inv = 1.0 / (d + 1e-16)
    out2 = jnp.concatenate([acc_ref[0], acc_ref[1]], axis=1)
    out2 = out2 * inv[:, None] + b2_ref[...][None, :]
    w3p = jnp.concatenate([w3_ref[...], jnp.zeros((D_MID, 14), _f32)], axis=1)
    a3s = jnp.concatenate([asrc_ref[...], jnp.zeros((14,), _f32)])
    a3d = jnp.concatenate([adst_ref[...], jnp.zeros((14,), _f32)])
    h3 = jnp.dot(out2, w3p, preferred_element_type=_f32)
    h3_ref[...] = h3
    s3_ref[...] = h3 @ a3s
    t3_ref[...] = h3 @ a3d


_tc_post2 = pl.pallas_call(
    _tc_post2_body,
    out_shape=[
        jax.ShapeDtypeStruct((NP, 16), _f32),
        jax.ShapeDtypeStruct((NP,), _f32),
        jax.ShapeDtypeStruct((NP,), _f32),
    ],
)


def _tc_post3_body(acc_ref, den_ref, b3_ref, wc_ref, bc_ref, out_ref):
    d = jnp.sum(den_ref[...], axis=0)
    inv = 1.0 / (d + 1e-16)
    out3 = (acc_ref[0] + acc_ref[1])[:, :2] * inv[:, None] + b3_ref[...][None, :]
    logits = jnp.dot(out3, wc_ref[...], preferred_element_type=_f32)
    logits = logits + bc_ref[...][None, :]
    row = lax.broadcasted_iota(_i32, (NP, 1), 0)
    valid = row < N
    neg = jnp.float32(-jnp.inf)
    masked = jnp.where(valid, logits, neg)
    mx = jnp.max(masked, axis=0, keepdims=True)
    z = jnp.where(valid, jnp.exp(logits - mx), 0.0)
    lse = jnp.log(jnp.sum(z, axis=0, keepdims=True)) + mx
    out_ref[...] = lax.slice(logits - lse, (0, 0), (N, 21))


_tc_post3 = pl.pallas_call(
    _tc_post3_body,
    out_shape=jax.ShapeDtypeStruct((N, 21), _f32),
)


def kernel(x, edge_index, edge_attr, W1, We1, a1_src, a1_dst, a1_edge, b1,
           W2, We2, a2_src, a2_dst, a2_edge, b2,
           W3, We3, a3_src, a3_dst, a3_edge, b3, Wc, bc):
    z1 = jnp.zeros((NP,), _f32)
    z16 = jnp.zeros((NP, 16), _f32)
    z64 = jnp.zeros((NP, 64), _f32)

    h1, s1, t1 = _tc_pre(x, W1, a1_src, a1_dst)
    src = jnp.pad(edge_index[0], (0, EP - E), constant_values=N)
    dst = jnp.pad(edge_index[1], (0, EP - E), constant_values=N)
    eg1, eg2, eg3 = [
        a.reshape(EP) for a in _tc_eg(
            edge_attr, We1, a1_edge, We2, a2_edge, We3, a3_edge)]

    acc1, den1 = _sc_layer16(src, dst, eg1, s1, t1, h1, z16, z1)
    h2, s2, t2 = _tc_post1(acc1, den1.reshape(NW, NP), b1, W2, a2_src, a2_dst)

    acc2, den2 = _sc_layer128(src, dst, eg2, s2, t2, h2, z64, z1)
    h3, s3, t3 = _tc_post2(acc2, den2, b2, W3, a3_src, a3_dst)

    acc3, den3 = _sc_layer16(src, dst, eg3, s3, t3, h3, z16, z1)
    return _tc_post3(acc3, den3.reshape(NW, NP), b3, Wc, bc)
